# baseline (speedup 1.0000x reference)
import functools

import jax
import jax.numpy as jnp
import numpy as np
from jax import lax
from jax.experimental import pallas as pl
from jax.experimental.pallas import tpu as pltpu
from jax.experimental.pallas import tpu_sc as plsc

_P = 2
_K = 24
_C = 64
_W = 128
_WP = _W + 2 * _P
_N_TB = 2 * _P * _WP
_N_LR = _W * 2 * _P
_N_HALO = _N_TB + _N_LR
_N_PAD = 1152
_ROWS = _N_PAD // 128
_NW = 32
_SLICES = _K * _C
_PER_W = _SLICES // _NW


_SLICE_WORDS = _WP * _WP


def _halo_dest_np():
    rows = np.full((_N_PAD,), _WP, dtype=np.int32)
    cols = np.arange(_N_PAD, dtype=np.int32) % 128
    s = np.arange(_N_TB)
    i, j = s // _WP, s % _WP
    rows[:_N_TB] = np.where(i < _P, i, _W + i)
    cols[:_N_TB] = j
    t = np.arange(_N_LR)
    r, c = t // (2 * _P), t % (2 * _P)
    rows[_N_TB:_N_HALO] = _P + r
    cols[_N_TB:_N_HALO] = np.where(c < _P, c, _W + c)
    return rows.reshape(_ROWS, 128), cols.reshape(_ROWS, 128)


_DEST_R, _DEST_C = _halo_dest_np()


def _halo_indices(to_process, batch_size):
    c, w, p = _C, _W, _P
    wp = _WP
    t = (2.0 * (jnp.arange(wp, dtype=jnp.float32) - p) + 1.0 - w) / w
    u = jnp.broadcast_to(t[None, :], (wp, wp))
    v = jnp.broadcast_to(t[:, None], (wp, wp))
    one = jnp.ones((wp, wp), dtype=jnp.float32)
    dirs = jnp.stack([
        jnp.stack([one, -v, -u], axis=-1),
        jnp.stack([-one, -v, u], axis=-1),
        jnp.stack([u, one, v], axis=-1),
        jnp.stack([u, -one, -v], axis=-1),
        jnp.stack([u, -v, one], axis=-1),
        jnp.stack([-u, -v, -one], axis=-1),
    ], axis=0)
    x, y, z = dirs[..., 0], dirs[..., 1], dirs[..., 2]
    ax, ay, az = jnp.abs(x), jnp.abs(y), jnp.abs(z)
    is_x = (ax >= ay) & (ax >= az)
    is_y = jnp.logical_and(~is_x, ay >= az)
    face = jnp.where(is_x, jnp.where(x > 0, 0, 1),
           jnp.where(is_y, jnp.where(y > 0, 2, 3),
                     jnp.where(z > 0, 4, 5)))
    a = jnp.maximum(jnp.maximum(ax, ay), az)
    uc = jnp.stack([-z, z, x, x, x, -x], axis=0) / a
    vc = jnp.stack([-y, -y, z, -z, -y, -y], axis=0) / a
    u2 = jnp.take_along_axis(uc, face[None], axis=0)[0]
    v2 = jnp.take_along_axis(vc, face[None], axis=0)[0]
    jj = jnp.clip(jnp.floor((u2 + 1.0) * 0.5 * w), 0, w - 1).astype(jnp.int32)
    ii = jnp.clip(jnp.floor((v2 + 1.0) * 0.5 * w), 0, w - 1).astype(jnp.int32)
    pix = ii * w + jj
    face = face.astype(jnp.int32)
    pix_tb = jnp.concatenate([pix[:, :p, :], pix[:, wp - p:, :]], axis=1)
    pix_lr = jnp.concatenate([pix[:, p:wp - p, :p], pix[:, p:wp - p, wp - p:]], axis=2)
    f_tb = jnp.concatenate([face[:, :p, :], face[:, wp - p:, :]], axis=1)
    f_lr = jnp.concatenate([face[:, p:wp - p, :p], face[:, p:wp - p, wp - p:]], axis=2)

    n_faces = to_process.shape[0]
    k = n_faces
    bs = n_faces // 6
    bs_delta = jnp.asarray(batch_size, dtype=jnp.int32) - bs
    inv = jnp.full((n_faces,), -1, dtype=jnp.int32)
    inv = inv.at[to_process].set(jnp.arange(k, dtype=jnp.int32))
    boff = 6 * (jnp.arange(bs, dtype=jnp.int32) + bs_delta)[:, None, None]
    ftb = (f_tb[None] + boff[:, :, None]).reshape(n_faces, 2 * p, wp)[to_process]
    ftb = inv[ftb]
    flr = (f_lr[None] + boff[:, :, None]).reshape(n_faces, w, 2 * p)[to_process]
    flr = inv[flr]
    ptb = pix_tb[to_process % 6]
    plr = pix_lr[to_process % 6]
    base_tb = ftb * (c * w * w) + ptb
    base_lr = flr * (c * w * w) + plr
    m_tb = (ftb >= 0)
    m_lr = (flr >= 0)
    base = jnp.concatenate([
        jnp.where(m_tb, base_tb, 0).reshape(n_faces, _N_TB),
        jnp.where(m_lr, base_lr, 0).reshape(n_faces, _N_LR),
        jnp.zeros((n_faces, _N_PAD - _N_HALO), jnp.int32),
    ], axis=1)
    mask = jnp.concatenate([
        m_tb.reshape(n_faces, _N_TB).astype(jnp.float32),
        m_lr.reshape(n_faces, _N_LR).astype(jnp.float32),
        jnp.zeros((n_faces, _N_PAD - _N_HALO), jnp.float32),
    ], axis=1)
    return base, mask


def _sc_body(idx_hbm, mask_hbm, cube_flat, dr_hbm, dc_hbm, out_hbm,
             idxv0, idxv1, hv0, hv1, mv0, mv1, cbuf0, cbuf1, buf0, buf1,
             drv, dcv,
             isem0, isem1, lsem0, lsem1, gsem0, gsem1, osem0, osem1):
    idxv = (idxv0, idxv1)
    hv = (hv0, hv1)
    mv = (mv0, mv1)
    cbuf = (cbuf0, cbuf1)
    buf = (buf0, buf1)
    isem = (isem0, isem1)
    lsem = (lsem0, lsem1)
    gsem = (gsem0, gsem1)
    osem = (osem0, osem1)
    cid = lax.axis_index("c")
    sid = lax.axis_index("s")
    wid = sid * 2 + cid
    base_s = wid * _PER_W
    pltpu.sync_copy(dr_hbm, drv)
    pltpu.sync_copy(dc_hbm, dcv)
    viota = lax.iota(jnp.int32, 16)
    n_iter = _PER_W // 2

    def kch(s):
        s = jnp.minimum(s, _SLICES - 1)
        k = s // _C
        return s, k, s - k * _C

    def start(slot, s):
        s, k, ch = kch(s)
        src_off = pl.multiple_of(s * (_W * _W), 8)
        pltpu.async_copy(idx_hbm.at[k, ch], idxv[slot], isem[slot])
        pltpu.async_copy(mask_hbm.at[k], mv[slot], lsem[slot])
        pltpu.async_copy(cube_flat.at[pl.ds(src_off, _W * _W)],
                         cbuf[slot], lsem[slot])

    def mid(slot):
        pltpu.make_async_copy(idx_hbm.at[0, 0], idxv[slot], isem[slot]).wait()
        for r in range(_ROWS):
            pltpu.async_copy(cube_flat.at[idxv[slot].at[r]],
                             hv[slot].at[r], gsem[slot])

    def finish(slot, s, t):
        s, k, ch = kch(s)
        pltpu.make_async_copy(mask_hbm.at[0], mv[slot], lsem[slot]).wait()
        pltpu.make_async_copy(cube_flat.at[pl.ds(0, _W * _W)],
                              cbuf[slot], lsem[slot]).wait()
        for r in range(_ROWS):
            pltpu.make_async_copy(cube_flat.at[pl.ds(0, 128)],
                                  hv[slot].at[r], gsem[slot]).wait()

        @pl.when(t > 0)
        def _wait_prev_write():
            pltpu.make_async_copy(buf[slot].at[pl.ds(0, _WP)],
                                  out_hbm.at[0, 0], osem[slot]).wait()

        for j in range(_W * _W // 16):
            r, c0 = j // 8, (j % 8) * 16
            rowv = jnp.full((16,), _P + r, jnp.int32)
            colv = viota + (_P + c0)
            plsc.store_scatter(buf[slot], [rowv, colv],
                               cbuf[slot][pl.ds(j * 16, 16)])
        for g in range(_ROWS * 8):
            r, c0 = g // 8, (g % 8) * 16
            val = hv[slot][r, pl.ds(c0, 16)] * mv[slot][r, pl.ds(c0, 16)]
            plsc.store_scatter(buf[slot], [drv[r, pl.ds(c0, 16)],
                                           dcv[r, pl.ds(c0, 16)]], val)
        pltpu.async_copy(buf[slot].at[pl.ds(0, _WP)],
                         out_hbm.at[k, ch], osem[slot])

    start(0, base_s)
    mid(0)

    def body(t, carry):
        s0 = base_s + 2 * t
        start(1, s0 + 1)
        finish(0, s0, t)
        mid(1)

        @pl.when(t < n_iter - 1)
        def _next():
            start(0, s0 + 2)
            mid(0)

        finish(1, s0 + 1, t)
        return carry

    lax.fori_loop(0, n_iter, body, 0)
    pltpu.make_async_copy(buf0.at[pl.ds(0, _WP)], out_hbm.at[0, 0],
                          osem0).wait()
    pltpu.make_async_copy(buf1.at[pl.ds(0, _WP)], out_hbm.at[0, 0],
                          osem1).wait()


@functools.partial(jax.jit, static_argnums=())
def _sc_pad(idx_full, mask, cube, dr, dc):
    mesh = plsc.VectorSubcoreMesh(core_axis_name="c", subcore_axis_name="s",
                                  num_cores=2, num_subcores=16)
    f = pl.kernel(
        _sc_body,
        out_type=jax.ShapeDtypeStruct((_K, _C, _WP, _WP), jnp.float32),
        mesh=mesh,
        scratch_types=[
            pltpu.VMEM((_ROWS, 128), jnp.int32),
            pltpu.VMEM((_ROWS, 128), jnp.int32),
            pltpu.VMEM((_ROWS, 128), jnp.float32),
            pltpu.VMEM((_ROWS, 128), jnp.float32),
            pltpu.VMEM((_ROWS, 128), jnp.float32),
            pltpu.VMEM((_ROWS, 128), jnp.float32),
            pltpu.VMEM((_W * _W,), jnp.float32),
            pltpu.VMEM((_W * _W,), jnp.float32),
            pltpu.VMEM((_WP + 1, _WP), jnp.float32),
            pltpu.VMEM((_WP + 1, _WP), jnp.float32),
            pltpu.VMEM((_ROWS, 128), jnp.int32),
            pltpu.VMEM((_ROWS, 128), jnp.int32),
            pltpu.SemaphoreType.DMA,
            pltpu.SemaphoreType.DMA,
            pltpu.SemaphoreType.DMA,
            pltpu.SemaphoreType.DMA,
            pltpu.SemaphoreType.DMA,
            pltpu.SemaphoreType.DMA,
            pltpu.SemaphoreType.DMA,
            pltpu.SemaphoreType.DMA,
        ],
        compiler_params=pltpu.CompilerParams(use_tc_tiling_on_sc=False,
                                             needs_layout_passes=False),
    )
    return f(idx_full, mask, cube.reshape(-1), dr, dc)


def kernel(cube, to_process, batch_size):
    base, mask = _halo_indices(to_process, batch_size)
    ch_off = (jnp.arange(_C, dtype=jnp.int32) * (_W * _W))[None, :, None]
    idx_full = (base[:, None, :] + ch_off).reshape(_K, _C, _ROWS, 128)
    mask = mask.reshape(_K, _ROWS, 128)
    return _sc_pad(idx_full, mask, cube,
                   jnp.asarray(_DEST_R), jnp.asarray(_DEST_C))


# speedup vs baseline: 1.7411x; 1.7411x over previous
import functools

import jax
import jax.numpy as jnp
import numpy as np
from jax import lax
from jax.experimental import pallas as pl
from jax.experimental.pallas import tpu as pltpu
from jax.experimental.pallas import tpu_sc as plsc

_P = 2
_K = 24
_C = 64
_W = 128
_WP = _W + 2 * _P
_N_TB = 2 * _P * _WP
_N_LR = _W * 2 * _P
_N_HALO = _N_TB + _N_LR
_N_PAD = 1152
_ROWS = _N_PAD // 128
_NW = 32
_SLICES = _K * _C
_PER_W = _SLICES // _NW


_SLICE_WORDS = _WP * _WP


def _halo_dest_np():
    rows = np.full((_N_PAD,), _WP, dtype=np.int32)
    cols = np.arange(_N_PAD, dtype=np.int32) % 128
    s = np.arange(_N_TB)
    i, j = s // _WP, s % _WP
    rows[:_N_TB] = np.where(i < _P, i, _W + i)
    cols[:_N_TB] = j
    t = np.arange(_N_LR)
    r, c = t // (2 * _P), t % (2 * _P)
    rows[_N_TB:_N_HALO] = _P + r
    cols[_N_TB:_N_HALO] = np.where(c < _P, c, _W + c)
    return rows.reshape(_ROWS, 128), cols.reshape(_ROWS, 128)


_DEST_R, _DEST_C = _halo_dest_np()


def _halo_indices(to_process, batch_size):
    c, w, p = _C, _W, _P
    wp = _WP
    t = (2.0 * (jnp.arange(wp, dtype=jnp.float32) - p) + 1.0 - w) / w
    u = jnp.broadcast_to(t[None, :], (wp, wp))
    v = jnp.broadcast_to(t[:, None], (wp, wp))
    one = jnp.ones((wp, wp), dtype=jnp.float32)
    dirs = jnp.stack([
        jnp.stack([one, -v, -u], axis=-1),
        jnp.stack([-one, -v, u], axis=-1),
        jnp.stack([u, one, v], axis=-1),
        jnp.stack([u, -one, -v], axis=-1),
        jnp.stack([u, -v, one], axis=-1),
        jnp.stack([-u, -v, -one], axis=-1),
    ], axis=0)
    x, y, z = dirs[..., 0], dirs[..., 1], dirs[..., 2]
    ax, ay, az = jnp.abs(x), jnp.abs(y), jnp.abs(z)
    is_x = (ax >= ay) & (ax >= az)
    is_y = jnp.logical_and(~is_x, ay >= az)
    face = jnp.where(is_x, jnp.where(x > 0, 0, 1),
           jnp.where(is_y, jnp.where(y > 0, 2, 3),
                     jnp.where(z > 0, 4, 5)))
    a = jnp.maximum(jnp.maximum(ax, ay), az)
    uc = jnp.stack([-z, z, x, x, x, -x], axis=0) / a
    vc = jnp.stack([-y, -y, z, -z, -y, -y], axis=0) / a
    u2 = jnp.take_along_axis(uc, face[None], axis=0)[0]
    v2 = jnp.take_along_axis(vc, face[None], axis=0)[0]
    jj = jnp.clip(jnp.floor((u2 + 1.0) * 0.5 * w), 0, w - 1).astype(jnp.int32)
    ii = jnp.clip(jnp.floor((v2 + 1.0) * 0.5 * w), 0, w - 1).astype(jnp.int32)
    pix = ii * w + jj
    face = face.astype(jnp.int32)
    pix_tb = jnp.concatenate([pix[:, :p, :], pix[:, wp - p:, :]], axis=1)
    pix_lr = jnp.concatenate([pix[:, p:wp - p, :p], pix[:, p:wp - p, wp - p:]], axis=2)
    f_tb = jnp.concatenate([face[:, :p, :], face[:, wp - p:, :]], axis=1)
    f_lr = jnp.concatenate([face[:, p:wp - p, :p], face[:, p:wp - p, wp - p:]], axis=2)

    n_faces = to_process.shape[0]
    k = n_faces
    bs = n_faces // 6
    bs_delta = jnp.asarray(batch_size, dtype=jnp.int32) - bs
    inv = jnp.full((n_faces,), -1, dtype=jnp.int32)
    inv = inv.at[to_process].set(jnp.arange(k, dtype=jnp.int32))
    boff = 6 * (jnp.arange(bs, dtype=jnp.int32) + bs_delta)[:, None, None]
    ftb = (f_tb[None] + boff[:, :, None]).reshape(n_faces, 2 * p, wp)[to_process]
    ftb = inv[ftb]
    flr = (f_lr[None] + boff[:, :, None]).reshape(n_faces, w, 2 * p)[to_process]
    flr = inv[flr]
    ptb = pix_tb[to_process % 6]
    plr = pix_lr[to_process % 6]
    base_tb = ftb * (c * w * w) + ptb
    base_lr = flr * (c * w * w) + plr
    m_tb = (ftb >= 0)
    m_lr = (flr >= 0)
    base = jnp.concatenate([
        jnp.where(m_tb, base_tb, 0).reshape(n_faces, _N_TB),
        jnp.where(m_lr, base_lr, 0).reshape(n_faces, _N_LR),
        jnp.zeros((n_faces, _N_PAD - _N_HALO), jnp.int32),
    ], axis=1)
    mask = jnp.concatenate([
        m_tb.reshape(n_faces, _N_TB).astype(jnp.float32),
        m_lr.reshape(n_faces, _N_LR).astype(jnp.float32),
        jnp.zeros((n_faces, _N_PAD - _N_HALO), jnp.float32),
    ], axis=1)
    return base, mask


def _sc_body(idx_hbm, mask_hbm, cube_flat, dr_hbm, dc_hbm, out_hbm,
             idxv0, idxv1, hv0, hv1, mv0, mv1, cbuf0, cbuf1, buf0, buf1,
             drv, dcv,
             isem0, isem1, lsem0, lsem1, gsem0, gsem1, osem0, osem1):
    idxv = (idxv0, idxv1)
    hv = (hv0, hv1)
    mv = (mv0, mv1)
    cbuf = (cbuf0, cbuf1)
    buf = (buf0, buf1)
    isem = (isem0, isem1)
    lsem = (lsem0, lsem1)
    gsem = (gsem0, gsem1)
    osem = (osem0, osem1)
    cid = lax.axis_index("c")
    sid = lax.axis_index("s")
    wid = sid * 2 + cid
    base_s = wid * _PER_W
    pltpu.sync_copy(dr_hbm, drv)
    pltpu.sync_copy(dc_hbm, dcv)
    viota = lax.iota(jnp.int32, 16)
    n_iter = _PER_W // 2

    def kch(s):
        s = jnp.minimum(s, _SLICES - 1)
        k = s // _C
        return s, k, s - k * _C

    def start(slot, s):
        s, k, ch = kch(s)
        src_off = pl.multiple_of(s * (_W * _W), 8)
        pltpu.async_copy(idx_hbm.at[k, ch], idxv[slot], isem[slot])
        pltpu.async_copy(mask_hbm.at[k], mv[slot], lsem[slot])
        pltpu.async_copy(cube_flat.at[pl.ds(src_off, _W * _W)],
                         cbuf[slot], lsem[slot])

    def mid(slot):
        pltpu.make_async_copy(idx_hbm.at[0, 0], idxv[slot], isem[slot]).wait()
        for r in range(_ROWS):
            pltpu.async_copy(cube_flat.at[idxv[slot].at[r]],
                             hv[slot].at[r], gsem[slot])

    def finish(slot, s, t):
        s, k, ch = kch(s)
        pltpu.make_async_copy(mask_hbm.at[0], mv[slot], lsem[slot]).wait()
        pltpu.make_async_copy(cube_flat.at[pl.ds(0, _W * _W)],
                              cbuf[slot], lsem[slot]).wait()
        for r in range(_ROWS):
            pltpu.make_async_copy(cube_flat.at[pl.ds(0, 128)],
                                  hv[slot].at[r], gsem[slot]).wait()

        @pl.when(t > 0)
        def _wait_prev_write():
            pltpu.make_async_copy(buf[slot], out_hbm.at[0, 0],
                                  osem[slot]).wait()

        def int_row(r, carry):
            rowv = jnp.zeros((16,), jnp.int32) + (_P + r)
            for jj in range(8):
                colv = viota + (_P + jj * 16)
                plsc.store_scatter(buf[slot], [rowv, colv],
                                   cbuf[slot][pl.ds(r * 128 + jj * 16, 16)])
            return carry

        lax.fori_loop(0, _W, int_row, 0)
        for g in range(_N_HALO // 16):
            r, c0 = g // 8, (g % 8) * 16
            val = hv[slot][r, pl.ds(c0, 16)] * mv[slot][r, pl.ds(c0, 16)]
            plsc.store_scatter(buf[slot], [drv[r, pl.ds(c0, 16)],
                                           dcv[r, pl.ds(c0, 16)]], val)
        pltpu.async_copy(buf[slot], out_hbm.at[k, ch], osem[slot])

    start(0, base_s)
    mid(0)

    def body(t, carry):
        s0 = base_s + 2 * t
        start(1, s0 + 1)
        finish(0, s0, t)
        mid(1)

        @pl.when(t < n_iter - 1)
        def _next():
            start(0, s0 + 2)
            mid(0)

        finish(1, s0 + 1, t)
        return carry

    lax.fori_loop(0, n_iter, body, 0)
    pltpu.make_async_copy(buf0, out_hbm.at[0, 0], osem0).wait()
    pltpu.make_async_copy(buf1, out_hbm.at[0, 0], osem1).wait()


@functools.partial(jax.jit, static_argnums=())
def _sc_pad(idx_full, mask, cube, dr, dc):
    mesh = plsc.VectorSubcoreMesh(core_axis_name="c", subcore_axis_name="s",
                                  num_cores=2, num_subcores=16)
    f = pl.kernel(
        _sc_body,
        out_type=jax.ShapeDtypeStruct((_K, _C, _WP, _WP), jnp.float32),
        mesh=mesh,
        scratch_types=[
            pltpu.VMEM((_ROWS, 128), jnp.int32),
            pltpu.VMEM((_ROWS, 128), jnp.int32),
            pltpu.VMEM((_ROWS, 128), jnp.float32),
            pltpu.VMEM((_ROWS, 128), jnp.float32),
            pltpu.VMEM((_ROWS, 128), jnp.float32),
            pltpu.VMEM((_ROWS, 128), jnp.float32),
            pltpu.VMEM((_W * _W,), jnp.float32),
            pltpu.VMEM((_W * _W,), jnp.float32),
            pltpu.VMEM((_WP, _WP), jnp.float32),
            pltpu.VMEM((_WP, _WP), jnp.float32),
            pltpu.VMEM((_ROWS, 128), jnp.int32),
            pltpu.VMEM((_ROWS, 128), jnp.int32),
            pltpu.SemaphoreType.DMA,
            pltpu.SemaphoreType.DMA,
            pltpu.SemaphoreType.DMA,
            pltpu.SemaphoreType.DMA,
            pltpu.SemaphoreType.DMA,
            pltpu.SemaphoreType.DMA,
            pltpu.SemaphoreType.DMA,
            pltpu.SemaphoreType.DMA,
        ],
        compiler_params=pltpu.CompilerParams(use_tc_tiling_on_sc=True,
                                             needs_layout_passes=False),
    )
    return f(idx_full, mask, cube.reshape(-1), dr, dc)


def kernel(cube, to_process, batch_size):
    base, mask = _halo_indices(to_process, batch_size)
    ch_off = (jnp.arange(_C, dtype=jnp.int32) * (_W * _W))[None, :, None]
    idx_full = (base[:, None, :] + ch_off).reshape(_K, _C, _ROWS, 128)
    mask = mask.reshape(_K, _ROWS, 128)
    return _sc_pad(idx_full, mask, cube,
                   jnp.asarray(_DEST_R), jnp.asarray(_DEST_C))


# speedup vs baseline: 2.1105x; 1.2122x over previous
import functools

import jax
import jax.numpy as jnp
import numpy as np
from jax import lax
from jax.experimental import pallas as pl
from jax.experimental.pallas import tpu as pltpu
from jax.experimental.pallas import tpu_sc as plsc

_P = 2
_K = 24
_C = 64
_W = 128
_WP = _W + 2 * _P
_N_TB = 2 * _P * _WP
_N_LR = _W * 2 * _P
_N_HALO = _N_TB + _N_LR
_N_PAD = 1152
_ROWS = _N_PAD // 128
_NW = 32
_SLICES = _K * _C
_PER_W = _SLICES // _NW
_CB = 8


def _strip_dest_np():
    rows = np.zeros((_N_PAD,), dtype=np.int32)
    cols = np.arange(_N_PAD, dtype=np.int32) % 128
    s = np.arange(_N_TB)
    rows[:_N_TB] = s // _WP
    cols[:_N_TB] = s % _WP
    t = np.arange(_N_LR)
    rows[_N_TB:_N_HALO] = 4 + t % (2 * _P)
    cols[_N_TB:_N_HALO] = t // (2 * _P)
    return rows.reshape(_ROWS, 128), cols.reshape(_ROWS, 128)


_SDEST_R, _SDEST_C = _strip_dest_np()


def _take24(table, idx):
    out = jnp.zeros(idx.shape + table.shape[1:], table.dtype)
    for d in range(24):
        sel = (idx == d)
        sel = sel.reshape(sel.shape + (1,) * (table.ndim - 1))
        out = jnp.where(sel, table[d][None], out)
    return out


def _halo_indices(to_process, batch_size):
    c, w, p = _C, _W, _P
    wp = _WP
    t = (2.0 * (jnp.arange(wp, dtype=jnp.float32) - p) + 1.0 - w) / w
    u = jnp.broadcast_to(t[None, :], (wp, wp))
    v = jnp.broadcast_to(t[:, None], (wp, wp))
    one = jnp.ones((wp, wp), dtype=jnp.float32)
    dirs = jnp.stack([
        jnp.stack([one, -v, -u], axis=-1),
        jnp.stack([-one, -v, u], axis=-1),
        jnp.stack([u, one, v], axis=-1),
        jnp.stack([u, -one, -v], axis=-1),
        jnp.stack([u, -v, one], axis=-1),
        jnp.stack([-u, -v, -one], axis=-1),
    ], axis=0)
    x, y, z = dirs[..., 0], dirs[..., 1], dirs[..., 2]
    ax, ay, az = jnp.abs(x), jnp.abs(y), jnp.abs(z)
    is_x = (ax >= ay) & (ax >= az)
    is_y = jnp.logical_and(~is_x, ay >= az)
    face = jnp.where(is_x, jnp.where(x > 0, 0, 1),
           jnp.where(is_y, jnp.where(y > 0, 2, 3),
                     jnp.where(z > 0, 4, 5)))
    a = jnp.maximum(jnp.maximum(ax, ay), az)
    uc = jnp.stack([-z, z, x, x, x, -x], axis=0) / a
    vc = jnp.stack([-y, -y, z, -z, -y, -y], axis=0) / a
    u2 = jnp.zeros((6, wp, wp), jnp.float32)
    v2 = jnp.zeros((6, wp, wp), jnp.float32)
    for d in range(6):
        u2 = jnp.where(face == d, uc[d], u2)
        v2 = jnp.where(face == d, vc[d], v2)
    jj = jnp.clip(jnp.floor((u2 + 1.0) * 0.5 * w), 0, w - 1).astype(jnp.int32)
    ii = jnp.clip(jnp.floor((v2 + 1.0) * 0.5 * w), 0, w - 1).astype(jnp.int32)
    pix = ii * w + jj
    face = face.astype(jnp.int32)
    pix_tb = jnp.concatenate([pix[:, :p, :], pix[:, wp - p:, :]], axis=1)
    pix_lr = jnp.concatenate([pix[:, p:wp - p, :p], pix[:, p:wp - p, wp - p:]], axis=2)
    f_tb = jnp.concatenate([face[:, :p, :], face[:, wp - p:, :]], axis=1)
    f_lr = jnp.concatenate([face[:, p:wp - p, :p], face[:, p:wp - p, wp - p:]], axis=2)

    n_faces = to_process.shape[0]
    bs = n_faces // 6
    bs_delta = jnp.asarray(batch_size, dtype=jnp.int32) - bs
    ar = jnp.arange(n_faces, dtype=jnp.int32)
    inv = jnp.full((n_faces,), -1, dtype=jnp.int32)
    for i in range(n_faces):
        inv = jnp.where(ar == to_process[i], i, inv)
    boff = 6 * (jnp.arange(bs, dtype=jnp.int32) + bs_delta)[:, None, None]
    ftb_all = (f_tb[None] + boff[:, :, None]).reshape(n_faces, _N_TB)
    flr_all = (f_lr[None] + boff[:, :, None]).reshape(n_faces, _N_LR)
    ftb = _take24(ftb_all, to_process)
    flr = _take24(flr_all, to_process)
    ftb_i = jnp.zeros_like(ftb)
    flr_i = jnp.zeros_like(flr)
    for d in range(n_faces):
        ftb_i = jnp.where(ftb == d, inv[d], ftb_i)
        flr_i = jnp.where(flr == d, inv[d], flr_i)
    tp6 = to_process % 6
    ptb = jnp.zeros((n_faces, _N_TB), jnp.int32)
    plr = jnp.zeros((n_faces, _N_LR), jnp.int32)
    pt_flat = pix_tb.reshape(6, _N_TB)
    pl_flat = pix_lr.reshape(6, _N_LR)
    for d in range(6):
        sel = (tp6 == d)[:, None]
        ptb = jnp.where(sel, pt_flat[d][None], ptb)
        plr = jnp.where(sel, pl_flat[d][None], plr)
    base_tb = ftb_i * (c * w * w) + ptb
    base_lr = flr_i * (c * w * w) + plr
    m_tb = (ftb_i >= 0)
    m_lr = (flr_i >= 0)
    base = jnp.concatenate([
        jnp.where(m_tb, base_tb, 0),
        jnp.where(m_lr, base_lr, 0),
        jnp.zeros((n_faces, _N_PAD - _N_HALO), jnp.int32),
    ], axis=1)
    mask = jnp.concatenate([
        m_tb.astype(jnp.float32),
        m_lr.astype(jnp.float32),
        jnp.zeros((n_faces, _N_PAD - _N_HALO), jnp.float32),
    ], axis=1)
    return base, mask


def _sc_body(bidx_hbm, mask_hbm, cube_flat, dr_hbm, dc_hbm, strips_hbm,
             iv0, iv1, hv0, hv1, mv0, mv1, sbuf0, sbuf1, drv, dcv,
             isem0, isem1, lsem0, lsem1, gsem0, gsem1, osem0, osem1):
    iv = (iv0, iv1)
    hv = (hv0, hv1)
    mv = (mv0, mv1)
    sbuf = (sbuf0, sbuf1)
    isem = (isem0, isem1)
    lsem = (lsem0, lsem1)
    gsem = (gsem0, gsem1)
    osem = (osem0, osem1)
    cid = lax.axis_index("c")
    sid = lax.axis_index("s")
    wid = sid * 2 + cid
    base_s = wid * _PER_W
    pltpu.sync_copy(dr_hbm, drv)
    pltpu.sync_copy(dc_hbm, dcv)
    n_iter = _PER_W // 2

    def kch(s):
        s = jnp.minimum(s, _SLICES - 1)
        k = s // _C
        return s, k, s - k * _C

    def start(slot, s):
        s, k, ch = kch(s)
        pltpu.async_copy(bidx_hbm.at[k], iv[slot], isem[slot])
        pltpu.async_copy(mask_hbm.at[k], mv[slot], lsem[slot])

    def mid(slot, s):
        s, k, ch = kch(s)
        pltpu.make_async_copy(bidx_hbm.at[0], iv[slot], isem[slot]).wait()
        choff = jnp.zeros((16,), jnp.int32) + ch * (_W * _W)
        for g in range(_ROWS * 8):
            r, c0 = g // 8, (g % 8) * 16
            iv[slot][r, pl.ds(c0, 16)] = iv[slot][r, pl.ds(c0, 16)] + choff
        for r in range(_ROWS):
            pltpu.async_copy(cube_flat.at[iv[slot].at[r]],
                             hv[slot].at[r], gsem[slot])

    def finish(slot, s, t):
        s, k, ch = kch(s)
        pltpu.make_async_copy(mask_hbm.at[0], mv[slot], lsem[slot]).wait()
        for r in range(_ROWS):
            pltpu.make_async_copy(cube_flat.at[pl.ds(0, 128)],
                                  hv[slot].at[r], gsem[slot]).wait()

        @pl.when(t > 0)
        def _wait_prev_write():
            pltpu.make_async_copy(sbuf[slot], strips_hbm.at[0, 0],
                                  osem[slot]).wait()

        for g in range(_N_HALO // 16):
            r, c0 = g // 8, (g % 8) * 16
            val = hv[slot][r, pl.ds(c0, 16)] * mv[slot][r, pl.ds(c0, 16)]
            plsc.store_scatter(sbuf[slot], [drv[r, pl.ds(c0, 16)],
                                            dcv[r, pl.ds(c0, 16)]], val)
        pltpu.async_copy(sbuf[slot], strips_hbm.at[k, ch], osem[slot])

    start(0, base_s)
    mid(0, base_s)

    def body(t, carry):
        s0 = base_s + 2 * t
        start(1, s0 + 1)
        finish(0, s0, t)
        mid(1, s0 + 1)

        @pl.when(t < n_iter - 1)
        def _next():
            start(0, s0 + 2)
            mid(0, s0 + 2)

        finish(1, s0 + 1, t)
        return carry

    lax.fori_loop(0, n_iter, body, 0)
    pltpu.make_async_copy(sbuf0, strips_hbm.at[0, 0], osem0).wait()
    pltpu.make_async_copy(sbuf1, strips_hbm.at[0, 0], osem1).wait()


def _tc_body(cube_ref, strip_ref, out_ref):
    for j in range(_CB):
        cb = cube_ref[0, j]
        st = strip_ref[0, j]
        left = jnp.transpose(st[4:6, 0:_W])
        right = jnp.transpose(st[6:8, 0:_W])
        mid = jnp.concatenate([left, cb, right], axis=1)
        out_ref[0, j] = jnp.concatenate(
            [st[0:2, :], mid, st[2:4, :]], axis=0)


@functools.partial(jax.jit, static_argnums=())
def _sc_tc_pad(bidx, mask, cube, dr, dc):
    mesh = plsc.VectorSubcoreMesh(core_axis_name="c", subcore_axis_name="s",
                                  num_cores=2, num_subcores=16)
    gather_f = pl.kernel(
        _sc_body,
        out_type=jax.ShapeDtypeStruct((_K, _C, 8, _WP), jnp.float32),
        mesh=mesh,
        scratch_types=[
            pltpu.VMEM((_ROWS, 128), jnp.int32),
            pltpu.VMEM((_ROWS, 128), jnp.int32),
            pltpu.VMEM((_ROWS, 128), jnp.float32),
            pltpu.VMEM((_ROWS, 128), jnp.float32),
            pltpu.VMEM((_ROWS, 128), jnp.float32),
            pltpu.VMEM((_ROWS, 128), jnp.float32),
            pltpu.VMEM((8, _WP), jnp.float32),
            pltpu.VMEM((8, _WP), jnp.float32),
            pltpu.VMEM((_ROWS, 128), jnp.int32),
            pltpu.VMEM((_ROWS, 128), jnp.int32),
            pltpu.SemaphoreType.DMA,
            pltpu.SemaphoreType.DMA,
            pltpu.SemaphoreType.DMA,
            pltpu.SemaphoreType.DMA,
            pltpu.SemaphoreType.DMA,
            pltpu.SemaphoreType.DMA,
            pltpu.SemaphoreType.DMA,
            pltpu.SemaphoreType.DMA,
        ],
        compiler_params=pltpu.CompilerParams(use_tc_tiling_on_sc=True,
                                             needs_layout_passes=False),
    )
    strips = gather_f(bidx, mask, cube.reshape(-1), dr, dc)
    asm = pl.pallas_call(
        _tc_body,
        grid=(_K, _C // _CB),
        in_specs=[
            pl.BlockSpec((1, _CB, _W, _W), lambda k, c: (k, c, 0, 0)),
            pl.BlockSpec((1, _CB, 8, _WP), lambda k, c: (k, c, 0, 0)),
        ],
        out_specs=pl.BlockSpec((1, _CB, _WP, _WP), lambda k, c: (k, c, 0, 0)),
        out_shape=jax.ShapeDtypeStruct((_K, _C, _WP, _WP), jnp.float32),
    )
    return asm(cube, strips)


def kernel(cube, to_process, batch_size):
    base, mask = _halo_indices(to_process, batch_size)
    return _sc_tc_pad(base.reshape(_K, _ROWS, 128),
                      mask.reshape(_K, _ROWS, 128), cube,
                      jnp.asarray(_SDEST_R), jnp.asarray(_SDEST_C))


# speedup vs baseline: 2.1949x; 1.0400x over previous
import functools

import jax
import jax.numpy as jnp
import numpy as np
from jax import lax
from jax.experimental import pallas as pl
from jax.experimental.pallas import tpu as pltpu
from jax.experimental.pallas import tpu_sc as plsc

_P = 2
_K = 24
_C = 64
_W = 128
_WP = _W + 2 * _P
_N_TB = 2 * _P * _WP
_N_LR = _W * 2 * _P
_N_HALO = _N_TB + _N_LR
_N_PAD = 1152
_ROWS = _N_PAD // 128
_NW = 32
_SLICES = _K * _C
_PER_W = _SLICES // _NW
_CB = 8


def _strip_dest_np():
    rows = np.zeros((_N_PAD,), dtype=np.int32)
    cols = np.arange(_N_PAD, dtype=np.int32) % 128
    s = np.arange(_N_TB)
    rows[:_N_TB] = s // _WP
    cols[:_N_TB] = s % _WP
    t = np.arange(_N_LR)
    rows[_N_TB:_N_HALO] = 4 + t % (2 * _P)
    cols[_N_TB:_N_HALO] = t // (2 * _P)
    return rows.reshape(_ROWS, 128), cols.reshape(_ROWS, 128)


_SDEST_R, _SDEST_C = _strip_dest_np()


def _take24(table, idx):
    out = jnp.zeros(idx.shape + table.shape[1:], table.dtype)
    for d in range(24):
        sel = (idx == d)
        sel = sel.reshape(sel.shape + (1,) * (table.ndim - 1))
        out = jnp.where(sel, table[d][None], out)
    return out


def _halo_indices(to_process, batch_size):
    c, w, p = _C, _W, _P
    wp = _WP
    t = (2.0 * (jnp.arange(wp, dtype=jnp.float32) - p) + 1.0 - w) / w
    u = jnp.broadcast_to(t[None, :], (wp, wp))
    v = jnp.broadcast_to(t[:, None], (wp, wp))
    one = jnp.ones((wp, wp), dtype=jnp.float32)
    dirs = jnp.stack([
        jnp.stack([one, -v, -u], axis=-1),
        jnp.stack([-one, -v, u], axis=-1),
        jnp.stack([u, one, v], axis=-1),
        jnp.stack([u, -one, -v], axis=-1),
        jnp.stack([u, -v, one], axis=-1),
        jnp.stack([-u, -v, -one], axis=-1),
    ], axis=0)
    x, y, z = dirs[..., 0], dirs[..., 1], dirs[..., 2]
    ax, ay, az = jnp.abs(x), jnp.abs(y), jnp.abs(z)
    is_x = (ax >= ay) & (ax >= az)
    is_y = jnp.logical_and(~is_x, ay >= az)
    face = jnp.where(is_x, jnp.where(x > 0, 0, 1),
           jnp.where(is_y, jnp.where(y > 0, 2, 3),
                     jnp.where(z > 0, 4, 5)))
    a = jnp.maximum(jnp.maximum(ax, ay), az)
    uc = jnp.stack([-z, z, x, x, x, -x], axis=0) / a
    vc = jnp.stack([-y, -y, z, -z, -y, -y], axis=0) / a
    u2 = jnp.zeros((6, wp, wp), jnp.float32)
    v2 = jnp.zeros((6, wp, wp), jnp.float32)
    for d in range(6):
        u2 = jnp.where(face == d, uc[d], u2)
        v2 = jnp.where(face == d, vc[d], v2)
    jj = jnp.clip(jnp.floor((u2 + 1.0) * 0.5 * w), 0, w - 1).astype(jnp.int32)
    ii = jnp.clip(jnp.floor((v2 + 1.0) * 0.5 * w), 0, w - 1).astype(jnp.int32)
    pix = ii * w + jj
    face = face.astype(jnp.int32)
    pix_tb = jnp.concatenate([pix[:, :p, :], pix[:, wp - p:, :]], axis=1)
    pix_lr = jnp.concatenate([pix[:, p:wp - p, :p], pix[:, p:wp - p, wp - p:]], axis=2)
    f_tb = jnp.concatenate([face[:, :p, :], face[:, wp - p:, :]], axis=1)
    f_lr = jnp.concatenate([face[:, p:wp - p, :p], face[:, p:wp - p, wp - p:]], axis=2)

    n_faces = to_process.shape[0]
    bs = n_faces // 6
    bs_delta = jnp.asarray(batch_size, dtype=jnp.int32) - bs
    ar = jnp.arange(n_faces, dtype=jnp.int32)
    inv = jnp.full((n_faces,), -1, dtype=jnp.int32)
    for i in range(n_faces):
        inv = jnp.where(ar == to_process[i], i, inv)
    boff = 6 * (jnp.arange(bs, dtype=jnp.int32) + bs_delta)[:, None, None]
    ftb_all = (f_tb[None] + boff[:, :, None]).reshape(n_faces, _N_TB)
    flr_all = (f_lr[None] + boff[:, :, None]).reshape(n_faces, _N_LR)
    ftb = _take24(ftb_all, to_process)
    flr = _take24(flr_all, to_process)
    ftb_i = jnp.zeros_like(ftb)
    flr_i = jnp.zeros_like(flr)
    for d in range(n_faces):
        ftb_i = jnp.where(ftb == d, inv[d], ftb_i)
        flr_i = jnp.where(flr == d, inv[d], flr_i)
    tp6 = to_process % 6
    ptb = jnp.zeros((n_faces, _N_TB), jnp.int32)
    plr = jnp.zeros((n_faces, _N_LR), jnp.int32)
    pt_flat = pix_tb.reshape(6, _N_TB)
    pl_flat = pix_lr.reshape(6, _N_LR)
    for d in range(6):
        sel = (tp6 == d)[:, None]
        ptb = jnp.where(sel, pt_flat[d][None], ptb)
        plr = jnp.where(sel, pl_flat[d][None], plr)
    base_tb = ftb_i * (c * w * w) + ptb
    base_lr = flr_i * (c * w * w) + plr
    m_tb = (ftb_i >= 0)
    m_lr = (flr_i >= 0)
    base = jnp.concatenate([
        jnp.where(m_tb, base_tb, 0),
        jnp.where(m_lr, base_lr, 0),
        jnp.zeros((n_faces, _N_PAD - _N_HALO), jnp.int32),
    ], axis=1)
    mk_tb = m_tb.astype(jnp.float32).reshape(n_faces, 4, _WP)
    mk_lr = jnp.transpose(m_lr.astype(jnp.float32).reshape(n_faces, _W, 4),
                          (0, 2, 1))
    mk_lr = jnp.pad(mk_lr, ((0, 0), (0, 0), (0, 2 * p)))
    mask_strip = jnp.concatenate([mk_tb, mk_lr], axis=1)
    return base, mask_strip


_R = 4


def _sc_body(bidx_hbm, cube_flat, dr_hbm, dc_hbm, strips_hbm,
             iv0, iv1, iv2, iv3, hv0, hv1, hv2, hv3,
             sbuf0, sbuf1, sbuf2, sbuf3, drv, dcv,
             isem0, isem1, isem2, isem3,
             gsem0, gsem1, gsem2, gsem3,
             osem0, osem1, osem2, osem3):
    iv = (iv0, iv1, iv2, iv3)
    hv = (hv0, hv1, hv2, hv3)
    sbuf = (sbuf0, sbuf1, sbuf2, sbuf3)
    isem = (isem0, isem1, isem2, isem3)
    gsem = (gsem0, gsem1, gsem2, gsem3)
    osem = (osem0, osem1, osem2, osem3)
    cid = lax.axis_index("c")
    sid = lax.axis_index("s")
    wid = sid * 2 + cid
    base_s = wid * _PER_W
    pltpu.sync_copy(dr_hbm, drv)
    pltpu.sync_copy(dc_hbm, dcv)
    n_iter = _PER_W // _R

    def kch(s):
        k = s // _C
        return k, s - k * _C

    def start(slot, s):
        k, ch = kch(s)
        pltpu.async_copy(bidx_hbm.at[k], iv[slot], isem[slot])

    def mid(slot, s):
        k, ch = kch(s)
        pltpu.make_async_copy(bidx_hbm.at[0], iv[slot], isem[slot]).wait()
        choff = jnp.zeros((16,), jnp.int32) + ch * (_W * _W)
        for g in range(_ROWS * 8):
            r, c0 = g // 8, (g % 8) * 16
            iv[slot][r, pl.ds(c0, 16)] = iv[slot][r, pl.ds(c0, 16)] + choff
        for r in range(_ROWS):
            pltpu.async_copy(cube_flat.at[iv[slot].at[r]],
                             hv[slot].at[r], gsem[slot])

    def finish(slot, s, t):
        k, ch = kch(s)
        for r in range(_ROWS):
            pltpu.make_async_copy(cube_flat.at[pl.ds(0, 128)],
                                  hv[slot].at[r], gsem[slot]).wait()

        @pl.when(t > 0)
        def _wait_prev_write():
            pltpu.make_async_copy(sbuf[slot], strips_hbm.at[0, 0],
                                  osem[slot]).wait()

        for g in range(_N_HALO // 16):
            r, c0 = g // 8, (g % 8) * 16
            plsc.store_scatter(sbuf[slot], [drv[r, pl.ds(c0, 16)],
                                            dcv[r, pl.ds(c0, 16)]],
                               hv[slot][r, pl.ds(c0, 16)])
        pltpu.async_copy(sbuf[slot], strips_hbm.at[k, ch], osem[slot])

    for r in range(_R - 1):
        start(r, base_s + r)
        mid(r, base_s + r)

    def body(t, carry):
        s0 = base_s + _R * t
        for r in range(_R):
            s = s0 + r
            sf = s + _R - 1
            slot_f = (r + _R - 1) % _R

            @pl.when(sf < base_s + _PER_W)
            def _fire():
                start(slot_f, sf)
                mid(slot_f, sf)

            finish(r, s, t)
        return carry

    lax.fori_loop(0, n_iter, body, 0)
    pltpu.make_async_copy(sbuf0, strips_hbm.at[0, 0], osem0).wait()
    pltpu.make_async_copy(sbuf1, strips_hbm.at[0, 0], osem1).wait()
    pltpu.make_async_copy(sbuf2, strips_hbm.at[0, 0], osem2).wait()
    pltpu.make_async_copy(sbuf3, strips_hbm.at[0, 0], osem3).wait()


def _tc_body(cube_ref, strip_ref, mask_ref, out_ref):
    mk = mask_ref[0]
    for j in range(_CB):
        cb = cube_ref[0, j]
        st = strip_ref[0, j] * mk
        left = jnp.transpose(st[4:6, 0:_W])
        right = jnp.transpose(st[6:8, 0:_W])
        mid = jnp.concatenate([left, cb, right], axis=1)
        out_ref[0, j] = jnp.concatenate(
            [st[0:2, :], mid, st[2:4, :]], axis=0)


@functools.partial(jax.jit, static_argnums=())
def _sc_tc_pad(bidx, mask_strip, cube, dr, dc):
    mesh = plsc.VectorSubcoreMesh(core_axis_name="c", subcore_axis_name="s",
                                  num_cores=2, num_subcores=16)
    gather_f = pl.kernel(
        _sc_body,
        out_type=jax.ShapeDtypeStruct((_K, _C, 8, _WP), jnp.float32),
        mesh=mesh,
        scratch_types=(
            [pltpu.VMEM((_ROWS, 128), jnp.int32) for _ in range(_R)] +
            [pltpu.VMEM((_ROWS, 128), jnp.float32) for _ in range(_R)] +
            [pltpu.VMEM((8, _WP), jnp.float32) for _ in range(_R)] +
            [pltpu.VMEM((_ROWS, 128), jnp.int32),
             pltpu.VMEM((_ROWS, 128), jnp.int32)] +
            [pltpu.SemaphoreType.DMA for _ in range(3 * _R)]
        ),
        compiler_params=pltpu.CompilerParams(use_tc_tiling_on_sc=True,
                                             needs_layout_passes=False),
    )
    strips = gather_f(bidx, cube.reshape(-1), dr, dc)
    asm = pl.pallas_call(
        _tc_body,
        grid=(_K, _C // _CB),
        in_specs=[
            pl.BlockSpec((1, _CB, _W, _W), lambda k, c: (k, c, 0, 0)),
            pl.BlockSpec((1, _CB, 8, _WP), lambda k, c: (k, c, 0, 0)),
            pl.BlockSpec((1, 8, _WP), lambda k, c: (k, 0, 0)),
        ],
        out_specs=pl.BlockSpec((1, _CB, _WP, _WP), lambda k, c: (k, c, 0, 0)),
        out_shape=jax.ShapeDtypeStruct((_K, _C, _WP, _WP), jnp.float32),
    )
    return asm(cube, strips, mask_strip)


def kernel(cube, to_process, batch_size):
    base, mask_strip = _halo_indices(to_process, batch_size)
    return _sc_tc_pad(base.reshape(_K, _ROWS, 128), mask_strip, cube,
                      jnp.asarray(_SDEST_R), jnp.asarray(_SDEST_C))


# speedup vs baseline: 2.4131x; 1.0994x over previous
import functools

import jax
import jax.numpy as jnp
import numpy as np
from jax import lax
from jax.experimental import pallas as pl
from jax.experimental.pallas import tpu as pltpu
from jax.experimental.pallas import tpu_sc as plsc

_P = 2
_K = 24
_C = 64
_W = 128
_WP = _W + 2 * _P
_N_TB = 2 * _P * _WP
_N_LR = _W * 2 * _P
_N_HALO = _N_TB + _N_LR
_N_PAD = 1152
_ROWS = _N_PAD // 128
_NW = 32
_SLICES = _K * _C
_PER_W = _SLICES // _NW
_CB = 16


def _strip_dest_np():
    rows = np.zeros((_N_PAD,), dtype=np.int32)
    cols = np.arange(_N_PAD, dtype=np.int32) % 128
    s = np.arange(_N_TB)
    rows[:_N_TB] = s // _WP
    cols[:_N_TB] = s % _WP
    t = np.arange(_N_LR)
    rows[_N_TB:_N_HALO] = 4 + t % (2 * _P)
    cols[_N_TB:_N_HALO] = t // (2 * _P)
    return rows.reshape(_ROWS, 128), cols.reshape(_ROWS, 128)


_SDEST_R, _SDEST_C = _strip_dest_np()


def _take24(table, idx):
    out = jnp.zeros(idx.shape + table.shape[1:], table.dtype)
    for d in range(24):
        sel = (idx == d)
        sel = sel.reshape(sel.shape + (1,) * (table.ndim - 1))
        out = jnp.where(sel, table[d][None], out)
    return out


def _halo_indices(to_process, batch_size):
    c, w, p = _C, _W, _P
    wp = _WP
    t = (2.0 * (jnp.arange(wp, dtype=jnp.float32) - p) + 1.0 - w) / w
    u = jnp.broadcast_to(t[None, :], (wp, wp))
    v = jnp.broadcast_to(t[:, None], (wp, wp))
    one = jnp.ones((wp, wp), dtype=jnp.float32)
    dirs = jnp.stack([
        jnp.stack([one, -v, -u], axis=-1),
        jnp.stack([-one, -v, u], axis=-1),
        jnp.stack([u, one, v], axis=-1),
        jnp.stack([u, -one, -v], axis=-1),
        jnp.stack([u, -v, one], axis=-1),
        jnp.stack([-u, -v, -one], axis=-1),
    ], axis=0)
    x, y, z = dirs[..., 0], dirs[..., 1], dirs[..., 2]
    ax, ay, az = jnp.abs(x), jnp.abs(y), jnp.abs(z)
    is_x = (ax >= ay) & (ax >= az)
    is_y = jnp.logical_and(~is_x, ay >= az)
    face = jnp.where(is_x, jnp.where(x > 0, 0, 1),
           jnp.where(is_y, jnp.where(y > 0, 2, 3),
                     jnp.where(z > 0, 4, 5)))
    a = jnp.maximum(jnp.maximum(ax, ay), az)
    uc = jnp.stack([-z, z, x, x, x, -x], axis=0) / a
    vc = jnp.stack([-y, -y, z, -z, -y, -y], axis=0) / a
    u2 = jnp.zeros((6, wp, wp), jnp.float32)
    v2 = jnp.zeros((6, wp, wp), jnp.float32)
    for d in range(6):
        u2 = jnp.where(face == d, uc[d], u2)
        v2 = jnp.where(face == d, vc[d], v2)
    jj = jnp.clip(jnp.floor((u2 + 1.0) * 0.5 * w), 0, w - 1).astype(jnp.int32)
    ii = jnp.clip(jnp.floor((v2 + 1.0) * 0.5 * w), 0, w - 1).astype(jnp.int32)
    pix = ii * w + jj
    face = face.astype(jnp.int32)
    pix_tb = jnp.concatenate([pix[:, :p, :], pix[:, wp - p:, :]], axis=1)
    pix_lr = jnp.concatenate([pix[:, p:wp - p, :p], pix[:, p:wp - p, wp - p:]], axis=2)
    f_tb = jnp.concatenate([face[:, :p, :], face[:, wp - p:, :]], axis=1)
    f_lr = jnp.concatenate([face[:, p:wp - p, :p], face[:, p:wp - p, wp - p:]], axis=2)

    n_faces = to_process.shape[0]
    bs = n_faces // 6
    bs_delta = jnp.asarray(batch_size, dtype=jnp.int32) - bs
    ar = jnp.arange(n_faces, dtype=jnp.int32)
    inv = jnp.full((n_faces,), -1, dtype=jnp.int32)
    for i in range(n_faces):
        inv = jnp.where(ar == to_process[i], i, inv)
    boff = 6 * (jnp.arange(bs, dtype=jnp.int32) + bs_delta)[:, None, None]
    ftb_all = (f_tb[None] + boff[:, :, None]).reshape(n_faces, _N_TB)
    flr_all = (f_lr[None] + boff[:, :, None]).reshape(n_faces, _N_LR)
    ftb = _take24(ftb_all, to_process)
    flr = _take24(flr_all, to_process)
    ftb_i = jnp.zeros_like(ftb)
    flr_i = jnp.zeros_like(flr)
    for d in range(n_faces):
        ftb_i = jnp.where(ftb == d, inv[d], ftb_i)
        flr_i = jnp.where(flr == d, inv[d], flr_i)
    tp6 = to_process % 6
    ptb = jnp.zeros((n_faces, _N_TB), jnp.int32)
    plr = jnp.zeros((n_faces, _N_LR), jnp.int32)
    pt_flat = pix_tb.reshape(6, _N_TB)
    pl_flat = pix_lr.reshape(6, _N_LR)
    for d in range(6):
        sel = (tp6 == d)[:, None]
        ptb = jnp.where(sel, pt_flat[d][None], ptb)
        plr = jnp.where(sel, pl_flat[d][None], plr)
    base_tb = ftb_i * (c * w * w) + ptb
    base_lr = flr_i * (c * w * w) + plr
    m_tb = (ftb_i >= 0)
    m_lr = (flr_i >= 0)
    base = jnp.concatenate([
        jnp.where(m_tb, base_tb, 0),
        jnp.where(m_lr, base_lr, 0),
        jnp.zeros((n_faces, _N_PAD - _N_HALO), jnp.int32),
    ], axis=1)
    mk_tb = m_tb.astype(jnp.float32).reshape(n_faces, 4, _WP)
    mk_lr = jnp.transpose(m_lr.astype(jnp.float32).reshape(n_faces, _W, 4),
                          (0, 2, 1))
    mk_lr = jnp.pad(mk_lr, ((0, 0), (0, 0), (0, 2 * p)))
    mask_strip = jnp.concatenate([mk_tb, mk_lr], axis=1)
    return base, mask_strip


_R = 4


def _sc_body(bidx_hbm, cube_flat, dr_hbm, dc_hbm, strips_hbm,
             iv0, iv1, iv2, iv3, hv0, hv1, hv2, hv3,
             sbuf0, sbuf1, sbuf2, sbuf3, drv, dcv,
             isem0, isem1, isem2, isem3,
             gsem0, gsem1, gsem2, gsem3,
             osem0, osem1, osem2, osem3):
    iv = (iv0, iv1, iv2, iv3)
    hv = (hv0, hv1, hv2, hv3)
    sbuf = (sbuf0, sbuf1, sbuf2, sbuf3)
    isem = (isem0, isem1, isem2, isem3)
    gsem = (gsem0, gsem1, gsem2, gsem3)
    osem = (osem0, osem1, osem2, osem3)
    cid = lax.axis_index("c")
    sid = lax.axis_index("s")
    wid = sid * 2 + cid
    base_s = wid * _PER_W
    pltpu.sync_copy(dr_hbm, drv)
    pltpu.sync_copy(dc_hbm, dcv)
    n_iter = _PER_W // _R

    def kch(s):
        k = s // _C
        return k, s - k * _C

    def start(slot, s):
        k, ch = kch(s)
        pltpu.async_copy(bidx_hbm.at[k], iv[slot], isem[slot])

    def mid(slot, s):
        k, ch = kch(s)
        pltpu.make_async_copy(bidx_hbm.at[0], iv[slot], isem[slot]).wait()
        choff = jnp.zeros((16,), jnp.int32) + ch * (_W * _W)
        for g in range(_ROWS * 8):
            r, c0 = g // 8, (g % 8) * 16
            iv[slot][r, pl.ds(c0, 16)] = iv[slot][r, pl.ds(c0, 16)] + choff
        for r in range(_ROWS):
            pltpu.async_copy(cube_flat.at[iv[slot].at[r]],
                             hv[slot].at[r], gsem[slot])

    def finish(slot, s, t):
        k, ch = kch(s)
        for r in range(_ROWS):
            pltpu.make_async_copy(cube_flat.at[pl.ds(0, 128)],
                                  hv[slot].at[r], gsem[slot]).wait()

        @pl.when(t > 0)
        def _wait_prev_write():
            pltpu.make_async_copy(sbuf[slot], strips_hbm.at[0, 0],
                                  osem[slot]).wait()

        for g in range(_N_HALO // 16):
            r, c0 = g // 8, (g % 8) * 16
            plsc.store_scatter(sbuf[slot], [drv[r, pl.ds(c0, 16)],
                                            dcv[r, pl.ds(c0, 16)]],
                               hv[slot][r, pl.ds(c0, 16)])
        pltpu.async_copy(sbuf[slot], strips_hbm.at[k, ch], osem[slot])

    for r in range(_R - 1):
        start(r, base_s + r)
        mid(r, base_s + r)

    def body(t, carry):
        s0 = base_s + _R * t
        for r in range(_R):
            s = s0 + r
            sf = s + _R - 1
            slot_f = (r + _R - 1) % _R

            @pl.when(sf < base_s + _PER_W)
            def _fire():
                start(slot_f, sf)
                mid(slot_f, sf)

            finish(r, s, t)
        return carry

    lax.fori_loop(0, n_iter, body, 0)
    pltpu.make_async_copy(sbuf0, strips_hbm.at[0, 0], osem0).wait()
    pltpu.make_async_copy(sbuf1, strips_hbm.at[0, 0], osem1).wait()
    pltpu.make_async_copy(sbuf2, strips_hbm.at[0, 0], osem2).wait()
    pltpu.make_async_copy(sbuf3, strips_hbm.at[0, 0], osem3).wait()


def _tc_body(cube_ref, strip_ref, mask_ref, out_ref):
    mk = mask_ref[0]
    for j in range(_CB):
        cb = cube_ref[0, j]
        st = strip_ref[0, j] * mk
        left = jnp.transpose(st[4:6, 0:_W])
        right = jnp.transpose(st[6:8, 0:_W])
        mid = jnp.concatenate([left, cb, right], axis=1)
        out_ref[0, j] = jnp.concatenate(
            [st[0:2, :], mid, st[2:4, :]], axis=0)


@functools.partial(jax.jit, static_argnums=())
def _sc_tc_pad(bidx, mask_strip, cube, dr, dc):
    mesh = plsc.VectorSubcoreMesh(core_axis_name="c", subcore_axis_name="s",
                                  num_cores=2, num_subcores=16)
    gather_f = pl.kernel(
        _sc_body,
        out_type=jax.ShapeDtypeStruct((_K, _C, 8, _WP), jnp.float32),
        mesh=mesh,
        scratch_types=(
            [pltpu.VMEM((_ROWS, 128), jnp.int32) for _ in range(_R)] +
            [pltpu.VMEM((_ROWS, 128), jnp.float32) for _ in range(_R)] +
            [pltpu.VMEM((8, _WP), jnp.float32) for _ in range(_R)] +
            [pltpu.VMEM((_ROWS, 128), jnp.int32),
             pltpu.VMEM((_ROWS, 128), jnp.int32)] +
            [pltpu.SemaphoreType.DMA for _ in range(3 * _R)]
        ),
        compiler_params=pltpu.CompilerParams(use_tc_tiling_on_sc=True,
                                             needs_layout_passes=False),
    )
    strips = gather_f(bidx, cube.reshape(-1), dr, dc)
    asm = pl.pallas_call(
        _tc_body,
        grid=(_K, _C // _CB),
        in_specs=[
            pl.BlockSpec((1, _CB, _W, _W), lambda k, c: (k, c, 0, 0)),
            pl.BlockSpec((1, _CB, 8, _WP), lambda k, c: (k, c, 0, 0)),
            pl.BlockSpec((1, 8, _WP), lambda k, c: (k, 0, 0)),
        ],
        out_specs=pl.BlockSpec((1, _CB, _WP, _WP), lambda k, c: (k, c, 0, 0)),
        out_shape=jax.ShapeDtypeStruct((_K, _C, _WP, _WP), jnp.float32),
    )
    return asm(cube, strips, mask_strip)


def kernel(cube, to_process, batch_size):
    base, mask_strip = _halo_indices(to_process, batch_size)
    return _sc_tc_pad(base.reshape(_K, _ROWS, 128), mask_strip, cube,
                      jnp.asarray(_SDEST_R), jnp.asarray(_SDEST_C))


# speedup vs baseline: 2.5430x; 1.0538x over previous
import functools

import jax
import jax.numpy as jnp
import numpy as np
from jax import lax
from jax.experimental import pallas as pl
from jax.experimental.pallas import tpu as pltpu
from jax.experimental.pallas import tpu_sc as plsc

_P = 2
_K = 24
_C = 64
_W = 128
_WP = _W + 2 * _P
_N_TB = 2 * _P * _WP
_N_LR = _W * 2 * _P
_N_HALO = _N_TB + _N_LR
_N_PAD = 1152
_ROWS = _N_PAD // 128
_NW = 32
_SLICES = _K * _C
_PER_W = _SLICES // _NW
_CB = 32


def _strip_dest_np():
    rows = np.zeros((_N_PAD,), dtype=np.int32)
    cols = np.arange(_N_PAD, dtype=np.int32) % 128
    s = np.arange(_N_TB)
    rows[:_N_TB] = s // _WP
    cols[:_N_TB] = s % _WP
    t = np.arange(_N_LR)
    rows[_N_TB:_N_HALO] = 4 + t % (2 * _P)
    cols[_N_TB:_N_HALO] = t // (2 * _P)
    return rows.reshape(_ROWS, 128), cols.reshape(_ROWS, 128)


_SDEST_R, _SDEST_C = _strip_dest_np()


def _take24(table, idx):
    out = jnp.zeros(idx.shape + table.shape[1:], table.dtype)
    for d in range(24):
        sel = (idx == d)
        sel = sel.reshape(sel.shape + (1,) * (table.ndim - 1))
        out = jnp.where(sel, table[d][None], out)
    return out


def _halo_indices(to_process, batch_size):
    c, w, p = _C, _W, _P
    wp = _WP
    t = (2.0 * (jnp.arange(wp, dtype=jnp.float32) - p) + 1.0 - w) / w
    u = jnp.broadcast_to(t[None, :], (wp, wp))
    v = jnp.broadcast_to(t[:, None], (wp, wp))
    one = jnp.ones((wp, wp), dtype=jnp.float32)
    dirs = jnp.stack([
        jnp.stack([one, -v, -u], axis=-1),
        jnp.stack([-one, -v, u], axis=-1),
        jnp.stack([u, one, v], axis=-1),
        jnp.stack([u, -one, -v], axis=-1),
        jnp.stack([u, -v, one], axis=-1),
        jnp.stack([-u, -v, -one], axis=-1),
    ], axis=0)
    x, y, z = dirs[..., 0], dirs[..., 1], dirs[..., 2]
    ax, ay, az = jnp.abs(x), jnp.abs(y), jnp.abs(z)
    is_x = (ax >= ay) & (ax >= az)
    is_y = jnp.logical_and(~is_x, ay >= az)
    face = jnp.where(is_x, jnp.where(x > 0, 0, 1),
           jnp.where(is_y, jnp.where(y > 0, 2, 3),
                     jnp.where(z > 0, 4, 5)))
    a = jnp.maximum(jnp.maximum(ax, ay), az)
    uc = jnp.stack([-z, z, x, x, x, -x], axis=0) / a
    vc = jnp.stack([-y, -y, z, -z, -y, -y], axis=0) / a
    u2 = jnp.zeros((6, wp, wp), jnp.float32)
    v2 = jnp.zeros((6, wp, wp), jnp.float32)
    for d in range(6):
        u2 = jnp.where(face == d, uc[d], u2)
        v2 = jnp.where(face == d, vc[d], v2)
    jj = jnp.clip(jnp.floor((u2 + 1.0) * 0.5 * w), 0, w - 1).astype(jnp.int32)
    ii = jnp.clip(jnp.floor((v2 + 1.0) * 0.5 * w), 0, w - 1).astype(jnp.int32)
    pix = ii * w + jj
    face = face.astype(jnp.int32)
    pix_tb = jnp.concatenate([pix[:, :p, :], pix[:, wp - p:, :]], axis=1)
    pix_lr = jnp.concatenate([pix[:, p:wp - p, :p], pix[:, p:wp - p, wp - p:]], axis=2)
    f_tb = jnp.concatenate([face[:, :p, :], face[:, wp - p:, :]], axis=1)
    f_lr = jnp.concatenate([face[:, p:wp - p, :p], face[:, p:wp - p, wp - p:]], axis=2)

    n_faces = to_process.shape[0]
    bs = n_faces // 6
    bs_delta = jnp.asarray(batch_size, dtype=jnp.int32) - bs
    ar = jnp.arange(n_faces, dtype=jnp.int32)
    inv = jnp.full((n_faces,), -1, dtype=jnp.int32)
    for i in range(n_faces):
        inv = jnp.where(ar == to_process[i], i, inv)
    boff = 6 * (jnp.arange(bs, dtype=jnp.int32) + bs_delta)[:, None, None]
    ftb_all = (f_tb[None] + boff[:, :, None]).reshape(n_faces, _N_TB)
    flr_all = (f_lr[None] + boff[:, :, None]).reshape(n_faces, _N_LR)
    ftb = _take24(ftb_all, to_process)
    flr = _take24(flr_all, to_process)
    ftb_i = jnp.zeros_like(ftb)
    flr_i = jnp.zeros_like(flr)
    for d in range(n_faces):
        ftb_i = jnp.where(ftb == d, inv[d], ftb_i)
        flr_i = jnp.where(flr == d, inv[d], flr_i)
    tp6 = to_process % 6
    ptb = jnp.zeros((n_faces, _N_TB), jnp.int32)
    plr = jnp.zeros((n_faces, _N_LR), jnp.int32)
    pt_flat = pix_tb.reshape(6, _N_TB)
    pl_flat = pix_lr.reshape(6, _N_LR)
    for d in range(6):
        sel = (tp6 == d)[:, None]
        ptb = jnp.where(sel, pt_flat[d][None], ptb)
        plr = jnp.where(sel, pl_flat[d][None], plr)
    base_tb = ftb_i * (c * w * w) + ptb
    base_lr = flr_i * (c * w * w) + plr
    m_tb = (ftb_i >= 0)
    m_lr = (flr_i >= 0)
    base = jnp.concatenate([
        jnp.where(m_tb, base_tb, 0),
        jnp.where(m_lr, base_lr, 0),
        jnp.zeros((n_faces, _N_PAD - _N_HALO), jnp.int32),
    ], axis=1)
    mk_tb = m_tb.astype(jnp.float32).reshape(n_faces, 4, _WP)
    mk_lr = jnp.transpose(m_lr.astype(jnp.float32).reshape(n_faces, _W, 4),
                          (0, 2, 1))
    mk_lr = jnp.pad(mk_lr, ((0, 0), (0, 0), (0, 2 * p)))
    mask_strip = jnp.concatenate([mk_tb, mk_lr], axis=1)
    return base, mask_strip


_R = 4


def _sc_body(bidx_hbm, cube_flat, dr_hbm, dc_hbm, strips_hbm,
             iv0, iv1, iv2, iv3, hv0, hv1, hv2, hv3,
             sbuf0, sbuf1, sbuf2, sbuf3, drv, dcv,
             isem0, isem1, isem2, isem3,
             gsem0, gsem1, gsem2, gsem3,
             osem0, osem1, osem2, osem3):
    iv = (iv0, iv1, iv2, iv3)
    hv = (hv0, hv1, hv2, hv3)
    sbuf = (sbuf0, sbuf1, sbuf2, sbuf3)
    isem = (isem0, isem1, isem2, isem3)
    gsem = (gsem0, gsem1, gsem2, gsem3)
    osem = (osem0, osem1, osem2, osem3)
    cid = lax.axis_index("c")
    sid = lax.axis_index("s")
    wid = sid * 2 + cid
    base_s = wid * _PER_W
    pltpu.sync_copy(dr_hbm, drv)
    pltpu.sync_copy(dc_hbm, dcv)
    n_iter = _PER_W // _R

    def kch(s):
        k = s // _C
        return k, s - k * _C

    def start(slot, s):
        k, ch = kch(s)
        pltpu.async_copy(bidx_hbm.at[k], iv[slot], isem[slot])

    def mid(slot, s):
        k, ch = kch(s)
        pltpu.make_async_copy(bidx_hbm.at[0], iv[slot], isem[slot]).wait()
        choff = jnp.zeros((16,), jnp.int32) + ch * (_W * _W)
        for g in range(_ROWS * 8):
            r, c0 = g // 8, (g % 8) * 16
            iv[slot][r, pl.ds(c0, 16)] = iv[slot][r, pl.ds(c0, 16)] + choff
        for r in range(_ROWS):
            pltpu.async_copy(cube_flat.at[iv[slot].at[r]],
                             hv[slot].at[r], gsem[slot])

    def finish(slot, s, t):
        k, ch = kch(s)
        for r in range(_ROWS):
            pltpu.make_async_copy(cube_flat.at[pl.ds(0, 128)],
                                  hv[slot].at[r], gsem[slot]).wait()

        @pl.when(t > 0)
        def _wait_prev_write():
            pltpu.make_async_copy(sbuf[slot], strips_hbm.at[0, 0],
                                  osem[slot]).wait()

        for g in range(_N_HALO // 16):
            r, c0 = g // 8, (g % 8) * 16
            plsc.store_scatter(sbuf[slot], [drv[r, pl.ds(c0, 16)],
                                            dcv[r, pl.ds(c0, 16)]],
                               hv[slot][r, pl.ds(c0, 16)])
        pltpu.async_copy(sbuf[slot], strips_hbm.at[k, ch], osem[slot])

    for r in range(_R - 1):
        start(r, base_s + r)
        mid(r, base_s + r)

    def body(t, carry):
        s0 = base_s + _R * t
        for r in range(_R):
            s = s0 + r
            sf = s + _R - 1
            slot_f = (r + _R - 1) % _R

            @pl.when(sf < base_s + _PER_W)
            def _fire():
                start(slot_f, sf)
                mid(slot_f, sf)

            finish(r, s, t)
        return carry

    lax.fori_loop(0, n_iter, body, 0)
    pltpu.make_async_copy(sbuf0, strips_hbm.at[0, 0], osem0).wait()
    pltpu.make_async_copy(sbuf1, strips_hbm.at[0, 0], osem1).wait()
    pltpu.make_async_copy(sbuf2, strips_hbm.at[0, 0], osem2).wait()
    pltpu.make_async_copy(sbuf3, strips_hbm.at[0, 0], osem3).wait()


def _tc_body(cube_ref, strip_ref, mask_ref, out_ref):
    mk = mask_ref[0]
    for j in range(_CB):
        cb = cube_ref[0, j]
        st = strip_ref[0, j] * mk
        left = jnp.transpose(st[4:6, 0:_W])
        right = jnp.transpose(st[6:8, 0:_W])
        mid = jnp.concatenate([left, cb, right], axis=1)
        out_ref[0, j] = jnp.concatenate(
            [st[0:2, :], mid, st[2:4, :]], axis=0)


@functools.partial(jax.jit, static_argnums=())
def _sc_tc_pad(bidx, mask_strip, cube, dr, dc):
    mesh = plsc.VectorSubcoreMesh(core_axis_name="c", subcore_axis_name="s",
                                  num_cores=2, num_subcores=16)
    gather_f = pl.kernel(
        _sc_body,
        out_type=jax.ShapeDtypeStruct((_K, _C, 8, _WP), jnp.float32),
        mesh=mesh,
        scratch_types=(
            [pltpu.VMEM((_ROWS, 128), jnp.int32) for _ in range(_R)] +
            [pltpu.VMEM((_ROWS, 128), jnp.float32) for _ in range(_R)] +
            [pltpu.VMEM((8, _WP), jnp.float32) for _ in range(_R)] +
            [pltpu.VMEM((_ROWS, 128), jnp.int32),
             pltpu.VMEM((_ROWS, 128), jnp.int32)] +
            [pltpu.SemaphoreType.DMA for _ in range(3 * _R)]
        ),
        compiler_params=pltpu.CompilerParams(use_tc_tiling_on_sc=True,
                                             needs_layout_passes=False),
    )
    strips = gather_f(bidx, cube.reshape(-1), dr, dc)
    asm = pl.pallas_call(
        _tc_body,
        grid=(_K, _C // _CB),
        in_specs=[
            pl.BlockSpec((1, _CB, _W, _W), lambda k, c: (k, c, 0, 0)),
            pl.BlockSpec((1, _CB, 8, _WP), lambda k, c: (k, c, 0, 0)),
            pl.BlockSpec((1, 8, _WP), lambda k, c: (k, 0, 0)),
        ],
        out_specs=pl.BlockSpec((1, _CB, _WP, _WP), lambda k, c: (k, c, 0, 0)),
        out_shape=jax.ShapeDtypeStruct((_K, _C, _WP, _WP), jnp.float32),
    )
    return asm(cube, strips, mask_strip)


def kernel(cube, to_process, batch_size):
    base, mask_strip = _halo_indices(to_process, batch_size)
    return _sc_tc_pad(base.reshape(_K, _ROWS, 128), mask_strip, cube,
                      jnp.asarray(_SDEST_R), jnp.asarray(_SDEST_C))


# speedup vs baseline: 2.6266x; 1.0329x over previous
import functools

import jax
import jax.numpy as jnp
import numpy as np
from jax import lax
from jax.experimental import pallas as pl
from jax.experimental.pallas import tpu as pltpu
from jax.experimental.pallas import tpu_sc as plsc

_P = 2
_K = 24
_C = 64
_W = 128
_WP = _W + 2 * _P
_N_TB = 2 * _P * _WP
_N_LR = _W * 2 * _P
_N_HALO = _N_TB + _N_LR
_N_PAD = 1152
_ROWS = _N_PAD // 128
_NW = 32
_SLICES = _K * _C
_PER_W = _SLICES // _NW
_CB = 64


def _strip_dest_np():
    rows = np.zeros((_N_PAD,), dtype=np.int32)
    cols = np.arange(_N_PAD, dtype=np.int32) % 128
    s = np.arange(_N_TB)
    rows[:_N_TB] = s // _WP
    cols[:_N_TB] = s % _WP
    t = np.arange(_N_LR)
    rows[_N_TB:_N_HALO] = 4 + t % (2 * _P)
    cols[_N_TB:_N_HALO] = t // (2 * _P)
    return rows.reshape(_ROWS, 128), cols.reshape(_ROWS, 128)


_SDEST_R, _SDEST_C = _strip_dest_np()


def _take24(table, idx):
    out = jnp.zeros(idx.shape + table.shape[1:], table.dtype)
    for d in range(24):
        sel = (idx == d)
        sel = sel.reshape(sel.shape + (1,) * (table.ndim - 1))
        out = jnp.where(sel, table[d][None], out)
    return out


def _halo_indices(to_process, batch_size):
    c, w, p = _C, _W, _P
    wp = _WP
    t = (2.0 * (jnp.arange(wp, dtype=jnp.float32) - p) + 1.0 - w) / w
    u = jnp.broadcast_to(t[None, :], (wp, wp))
    v = jnp.broadcast_to(t[:, None], (wp, wp))
    one = jnp.ones((wp, wp), dtype=jnp.float32)
    dirs = jnp.stack([
        jnp.stack([one, -v, -u], axis=-1),
        jnp.stack([-one, -v, u], axis=-1),
        jnp.stack([u, one, v], axis=-1),
        jnp.stack([u, -one, -v], axis=-1),
        jnp.stack([u, -v, one], axis=-1),
        jnp.stack([-u, -v, -one], axis=-1),
    ], axis=0)
    x, y, z = dirs[..., 0], dirs[..., 1], dirs[..., 2]
    ax, ay, az = jnp.abs(x), jnp.abs(y), jnp.abs(z)
    is_x = (ax >= ay) & (ax >= az)
    is_y = jnp.logical_and(~is_x, ay >= az)
    face = jnp.where(is_x, jnp.where(x > 0, 0, 1),
           jnp.where(is_y, jnp.where(y > 0, 2, 3),
                     jnp.where(z > 0, 4, 5)))
    a = jnp.maximum(jnp.maximum(ax, ay), az)
    uc = jnp.stack([-z, z, x, x, x, -x], axis=0) / a
    vc = jnp.stack([-y, -y, z, -z, -y, -y], axis=0) / a
    u2 = jnp.zeros((6, wp, wp), jnp.float32)
    v2 = jnp.zeros((6, wp, wp), jnp.float32)
    for d in range(6):
        u2 = jnp.where(face == d, uc[d], u2)
        v2 = jnp.where(face == d, vc[d], v2)
    jj = jnp.clip(jnp.floor((u2 + 1.0) * 0.5 * w), 0, w - 1).astype(jnp.int32)
    ii = jnp.clip(jnp.floor((v2 + 1.0) * 0.5 * w), 0, w - 1).astype(jnp.int32)
    pix = ii * w + jj
    face = face.astype(jnp.int32)
    pix_tb = jnp.concatenate([pix[:, :p, :], pix[:, wp - p:, :]], axis=1)
    pix_lr = jnp.concatenate([pix[:, p:wp - p, :p], pix[:, p:wp - p, wp - p:]], axis=2)
    f_tb = jnp.concatenate([face[:, :p, :], face[:, wp - p:, :]], axis=1)
    f_lr = jnp.concatenate([face[:, p:wp - p, :p], face[:, p:wp - p, wp - p:]], axis=2)

    n_faces = to_process.shape[0]
    bs = n_faces // 6
    bs_delta = jnp.asarray(batch_size, dtype=jnp.int32) - bs
    ar = jnp.arange(n_faces, dtype=jnp.int32)
    inv = jnp.full((n_faces,), -1, dtype=jnp.int32)
    for i in range(n_faces):
        inv = jnp.where(ar == to_process[i], i, inv)
    boff = 6 * (jnp.arange(bs, dtype=jnp.int32) + bs_delta)[:, None, None]
    ftb_all = (f_tb[None] + boff[:, :, None]).reshape(n_faces, _N_TB)
    flr_all = (f_lr[None] + boff[:, :, None]).reshape(n_faces, _N_LR)
    ftb = _take24(ftb_all, to_process)
    flr = _take24(flr_all, to_process)
    ftb_i = jnp.zeros_like(ftb)
    flr_i = jnp.zeros_like(flr)
    for d in range(n_faces):
        ftb_i = jnp.where(ftb == d, inv[d], ftb_i)
        flr_i = jnp.where(flr == d, inv[d], flr_i)
    tp6 = to_process % 6
    ptb = jnp.zeros((n_faces, _N_TB), jnp.int32)
    plr = jnp.zeros((n_faces, _N_LR), jnp.int32)
    pt_flat = pix_tb.reshape(6, _N_TB)
    pl_flat = pix_lr.reshape(6, _N_LR)
    for d in range(6):
        sel = (tp6 == d)[:, None]
        ptb = jnp.where(sel, pt_flat[d][None], ptb)
        plr = jnp.where(sel, pl_flat[d][None], plr)
    base_tb = ftb_i * (c * w * w) + ptb
    base_lr = flr_i * (c * w * w) + plr
    m_tb = (ftb_i >= 0)
    m_lr = (flr_i >= 0)
    base = jnp.concatenate([
        jnp.where(m_tb, base_tb, 0),
        jnp.where(m_lr, base_lr, 0),
        jnp.zeros((n_faces, _N_PAD - _N_HALO), jnp.int32),
    ], axis=1)
    mk_tb = m_tb.astype(jnp.float32).reshape(n_faces, 4, _WP)
    mk_lr = jnp.transpose(m_lr.astype(jnp.float32).reshape(n_faces, _W, 4),
                          (0, 2, 1))
    mk_lr = jnp.pad(mk_lr, ((0, 0), (0, 0), (0, 2 * p)))
    mask_strip = jnp.concatenate([mk_tb, mk_lr], axis=1)
    return base, mask_strip


_R = 4


def _sc_body(bidx_hbm, cube_flat, dr_hbm, dc_hbm, strips_hbm,
             iv0, iv1, iv2, iv3, hv0, hv1, hv2, hv3,
             sbuf0, sbuf1, sbuf2, sbuf3, drv, dcv,
             isem0, isem1, isem2, isem3,
             gsem0, gsem1, gsem2, gsem3,
             osem0, osem1, osem2, osem3):
    iv = (iv0, iv1, iv2, iv3)
    hv = (hv0, hv1, hv2, hv3)
    sbuf = (sbuf0, sbuf1, sbuf2, sbuf3)
    isem = (isem0, isem1, isem2, isem3)
    gsem = (gsem0, gsem1, gsem2, gsem3)
    osem = (osem0, osem1, osem2, osem3)
    cid = lax.axis_index("c")
    sid = lax.axis_index("s")
    wid = sid * 2 + cid
    base_s = wid * _PER_W
    pltpu.sync_copy(dr_hbm, drv)
    pltpu.sync_copy(dc_hbm, dcv)
    n_iter = _PER_W // _R

    def kch(s):
        k = s // _C
        return k, s - k * _C

    def start(slot, s):
        k, ch = kch(s)
        pltpu.async_copy(bidx_hbm.at[k], iv[slot], isem[slot])

    def mid(slot, s):
        k, ch = kch(s)
        pltpu.make_async_copy(bidx_hbm.at[0], iv[slot], isem[slot]).wait()
        choff = jnp.zeros((16,), jnp.int32) + ch * (_W * _W)
        for g in range(_ROWS * 8):
            r, c0 = g // 8, (g % 8) * 16
            iv[slot][r, pl.ds(c0, 16)] = iv[slot][r, pl.ds(c0, 16)] + choff
        for r in range(_ROWS):
            pltpu.async_copy(cube_flat.at[iv[slot].at[r]],
                             hv[slot].at[r], gsem[slot])

    def finish(slot, s, t):
        k, ch = kch(s)
        for r in range(_ROWS):
            pltpu.make_async_copy(cube_flat.at[pl.ds(0, 128)],
                                  hv[slot].at[r], gsem[slot]).wait()

        @pl.when(t > 0)
        def _wait_prev_write():
            pltpu.make_async_copy(sbuf[slot], strips_hbm.at[0, 0],
                                  osem[slot]).wait()

        for g in range(_N_HALO // 16):
            r, c0 = g // 8, (g % 8) * 16
            plsc.store_scatter(sbuf[slot], [drv[r, pl.ds(c0, 16)],
                                            dcv[r, pl.ds(c0, 16)]],
                               hv[slot][r, pl.ds(c0, 16)])
        pltpu.async_copy(sbuf[slot], strips_hbm.at[k, ch], osem[slot])

    for r in range(_R - 1):
        start(r, base_s + r)
        mid(r, base_s + r)

    def body(t, carry):
        s0 = base_s + _R * t
        for r in range(_R):
            s = s0 + r
            sf = s + _R - 1
            slot_f = (r + _R - 1) % _R

            @pl.when(sf < base_s + _PER_W)
            def _fire():
                start(slot_f, sf)
                mid(slot_f, sf)

            finish(r, s, t)
        return carry

    lax.fori_loop(0, n_iter, body, 0)
    pltpu.make_async_copy(sbuf0, strips_hbm.at[0, 0], osem0).wait()
    pltpu.make_async_copy(sbuf1, strips_hbm.at[0, 0], osem1).wait()
    pltpu.make_async_copy(sbuf2, strips_hbm.at[0, 0], osem2).wait()
    pltpu.make_async_copy(sbuf3, strips_hbm.at[0, 0], osem3).wait()


def _tc_body(cube_ref, strip_ref, mask_ref, out_ref):
    mk = mask_ref[0]
    for j in range(_CB):
        cb = cube_ref[0, j]
        st = strip_ref[0, j] * mk
        left = jnp.transpose(st[4:6, 0:_W])
        right = jnp.transpose(st[6:8, 0:_W])
        mid = jnp.concatenate([left, cb, right], axis=1)
        out_ref[0, j] = jnp.concatenate(
            [st[0:2, :], mid, st[2:4, :]], axis=0)


@functools.partial(jax.jit, static_argnums=())
def _sc_tc_pad(bidx, mask_strip, cube, dr, dc):
    mesh = plsc.VectorSubcoreMesh(core_axis_name="c", subcore_axis_name="s",
                                  num_cores=2, num_subcores=16)
    gather_f = pl.kernel(
        _sc_body,
        out_type=jax.ShapeDtypeStruct((_K, _C, 8, _WP), jnp.float32),
        mesh=mesh,
        scratch_types=(
            [pltpu.VMEM((_ROWS, 128), jnp.int32) for _ in range(_R)] +
            [pltpu.VMEM((_ROWS, 128), jnp.float32) for _ in range(_R)] +
            [pltpu.VMEM((8, _WP), jnp.float32) for _ in range(_R)] +
            [pltpu.VMEM((_ROWS, 128), jnp.int32),
             pltpu.VMEM((_ROWS, 128), jnp.int32)] +
            [pltpu.SemaphoreType.DMA for _ in range(3 * _R)]
        ),
        compiler_params=pltpu.CompilerParams(use_tc_tiling_on_sc=True,
                                             needs_layout_passes=False),
    )
    strips = gather_f(bidx, cube.reshape(-1), dr, dc)
    asm = pl.pallas_call(
        _tc_body,
        grid=(_K, _C // _CB),
        in_specs=[
            pl.BlockSpec((1, _CB, _W, _W), lambda k, c: (k, c, 0, 0)),
            pl.BlockSpec((1, _CB, 8, _WP), lambda k, c: (k, c, 0, 0)),
            pl.BlockSpec((1, 8, _WP), lambda k, c: (k, 0, 0)),
        ],
        out_specs=pl.BlockSpec((1, _CB, _WP, _WP), lambda k, c: (k, c, 0, 0)),
        out_shape=jax.ShapeDtypeStruct((_K, _C, _WP, _WP), jnp.float32),
    )
    return asm(cube, strips, mask_strip)


def kernel(cube, to_process, batch_size):
    base, mask_strip = _halo_indices(to_process, batch_size)
    return _sc_tc_pad(base.reshape(_K, _ROWS, 128), mask_strip, cube,
                      jnp.asarray(_SDEST_R), jnp.asarray(_SDEST_C))


# speedup vs baseline: 2.7576x; 1.0499x over previous
import functools

import jax
import jax.numpy as jnp
import numpy as np
from jax import lax
from jax.experimental import pallas as pl
from jax.experimental.pallas import tpu as pltpu
from jax.experimental.pallas import tpu_sc as plsc

_P = 2
_K = 24
_C = 64
_W = 128
_WP = _W + 2 * _P
_N_TB = 2 * _P * _WP
_N_LR = _W * 2 * _P
_N_HALO = _N_TB + _N_LR
_N_PAD = 1152
_ROWS = _N_PAD // 128
_NW = 32
_SLICES = _K * _C
_PER_W = _SLICES // _NW
_CB = 64


def _strip_dest_np():
    rows = np.zeros((_N_PAD,), dtype=np.int32)
    cols = np.arange(_N_PAD, dtype=np.int32) % 128
    s = np.arange(_N_TB)
    rows[:_N_TB] = s // _WP
    cols[:_N_TB] = s % _WP
    t = np.arange(_N_LR)
    rows[_N_TB:_N_HALO] = 4 + t % (2 * _P)
    cols[_N_TB:_N_HALO] = t // (2 * _P)
    return rows.reshape(_ROWS, 128), cols.reshape(_ROWS, 128)


_SDEST_R, _SDEST_C = _strip_dest_np()


def _halo_indices(to_process, batch_size):
    c, w, p = _C, _W, _P
    wp = _WP
    t = (2.0 * (jnp.arange(wp, dtype=jnp.float32) - p) + 1.0 - w) / w
    u = jnp.broadcast_to(t[None, :], (wp, wp))
    v = jnp.broadcast_to(t[:, None], (wp, wp))
    one = jnp.ones((wp, wp), dtype=jnp.float32)
    dirs = jnp.stack([
        jnp.stack([one, -v, -u], axis=-1),
        jnp.stack([-one, -v, u], axis=-1),
        jnp.stack([u, one, v], axis=-1),
        jnp.stack([u, -one, -v], axis=-1),
        jnp.stack([u, -v, one], axis=-1),
        jnp.stack([-u, -v, -one], axis=-1),
    ], axis=0)
    x, y, z = dirs[..., 0], dirs[..., 1], dirs[..., 2]
    ax, ay, az = jnp.abs(x), jnp.abs(y), jnp.abs(z)
    is_x = (ax >= ay) & (ax >= az)
    is_y = jnp.logical_and(~is_x, ay >= az)
    face = jnp.where(is_x, jnp.where(x > 0, 0, 1),
           jnp.where(is_y, jnp.where(y > 0, 2, 3),
                     jnp.where(z > 0, 4, 5)))
    a = jnp.maximum(jnp.maximum(ax, ay), az)
    uc = jnp.stack([-z, z, x, x, x, -x], axis=0) / a
    vc = jnp.stack([-y, -y, z, -z, -y, -y], axis=0) / a
    u2 = jnp.zeros((6, wp, wp), jnp.float32)
    v2 = jnp.zeros((6, wp, wp), jnp.float32)
    for d in range(6):
        u2 = jnp.where(face == d, uc[d], u2)
        v2 = jnp.where(face == d, vc[d], v2)
    jj = jnp.clip(jnp.floor((u2 + 1.0) * 0.5 * w), 0, w - 1).astype(jnp.int32)
    ii = jnp.clip(jnp.floor((v2 + 1.0) * 0.5 * w), 0, w - 1).astype(jnp.int32)
    pix = ii * w + jj
    face = face.astype(jnp.int32)
    pix_tb = jnp.concatenate([pix[:, :p, :], pix[:, wp - p:, :]], axis=1)
    pix_lr = jnp.concatenate([pix[:, p:wp - p, :p], pix[:, p:wp - p, wp - p:]], axis=2)
    f_tb = jnp.concatenate([face[:, :p, :], face[:, wp - p:, :]], axis=1)
    f_lr = jnp.concatenate([face[:, p:wp - p, :p], face[:, p:wp - p, wp - p:]], axis=2)

    n_faces = to_process.shape[0]
    bs = n_faces // 6
    bs_delta = jnp.asarray(batch_size, dtype=jnp.int32) - bs
    ar = jnp.arange(n_faces, dtype=jnp.int32)
    inv = jnp.full((n_faces,), -1, dtype=jnp.int32)
    for i in range(n_faces):
        inv = jnp.where(ar == to_process[i], i, inv)
    tpm6 = to_process % 6
    tpd6 = to_process // 6
    idxmat = 6 * (tpd6 + bs_delta)[:, None] + jnp.arange(6, dtype=jnp.int32)[None, :]
    lut = jnp.zeros((n_faces, 6), jnp.int32)
    for d in range(n_faces):
        lut = jnp.where(idxmat == d, inv[d], lut)
    ft_flat = f_tb.reshape(6, _N_TB)
    fl_flat = f_lr.reshape(6, _N_LR)
    pt_flat = pix_tb.reshape(6, _N_TB)
    pl_flat = pix_lr.reshape(6, _N_LR)
    fsel_tb = jnp.zeros((n_faces, _N_TB), jnp.int32)
    fsel_lr = jnp.zeros((n_faces, _N_LR), jnp.int32)
    ptb = jnp.zeros((n_faces, _N_TB), jnp.int32)
    plr = jnp.zeros((n_faces, _N_LR), jnp.int32)
    for d in range(6):
        sel = (tpm6 == d)[:, None]
        fsel_tb = jnp.where(sel, ft_flat[d][None], fsel_tb)
        fsel_lr = jnp.where(sel, fl_flat[d][None], fsel_lr)
        ptb = jnp.where(sel, pt_flat[d][None], ptb)
        plr = jnp.where(sel, pl_flat[d][None], plr)
    ftb_i = jnp.zeros_like(fsel_tb)
    flr_i = jnp.zeros_like(fsel_lr)
    for j in range(6):
        ftb_i = jnp.where(fsel_tb == j, lut[:, j][:, None], ftb_i)
        flr_i = jnp.where(fsel_lr == j, lut[:, j][:, None], flr_i)
    base_tb = ftb_i * (c * w * w) + ptb
    base_lr = flr_i * (c * w * w) + plr
    m_tb = (ftb_i >= 0)
    m_lr = (flr_i >= 0)
    base = jnp.concatenate([
        jnp.where(m_tb, base_tb, 0),
        jnp.where(m_lr, base_lr, 0),
        jnp.zeros((n_faces, _N_PAD - _N_HALO), jnp.int32),
    ], axis=1)
    mk_tb = m_tb.astype(jnp.float32).reshape(n_faces, 4, _WP)
    mk_lr = jnp.transpose(m_lr.astype(jnp.float32).reshape(n_faces, _W, 4),
                          (0, 2, 1))
    mk_lr = jnp.pad(mk_lr, ((0, 0), (0, 0), (0, 2 * p)))
    mask_strip = jnp.concatenate([mk_tb, mk_lr], axis=1)
    return base, mask_strip


_R = 4


def _sc_body(bidx_hbm, cube_flat, dr_hbm, dc_hbm, strips_hbm,
             iv0, iv1, iv2, iv3, hv0, hv1, hv2, hv3,
             sbuf0, sbuf1, sbuf2, sbuf3, drv, dcv,
             isem0, isem1, isem2, isem3,
             gsem0, gsem1, gsem2, gsem3,
             osem0, osem1, osem2, osem3):
    iv = (iv0, iv1, iv2, iv3)
    hv = (hv0, hv1, hv2, hv3)
    sbuf = (sbuf0, sbuf1, sbuf2, sbuf3)
    isem = (isem0, isem1, isem2, isem3)
    gsem = (gsem0, gsem1, gsem2, gsem3)
    osem = (osem0, osem1, osem2, osem3)
    cid = lax.axis_index("c")
    sid = lax.axis_index("s")
    wid = sid * 2 + cid
    base_s = wid * _PER_W
    pltpu.sync_copy(dr_hbm, drv)
    pltpu.sync_copy(dc_hbm, dcv)
    n_iter = _PER_W // _R

    def kch(s):
        k = s // _C
        return k, s - k * _C

    def start(slot, s):
        k, ch = kch(s)
        pltpu.async_copy(bidx_hbm.at[k], iv[slot], isem[slot])

    def mid(slot, s):
        k, ch = kch(s)
        pltpu.make_async_copy(bidx_hbm.at[0], iv[slot], isem[slot]).wait()
        choff = jnp.zeros((16,), jnp.int32) + ch * (_W * _W)
        for g in range(_ROWS * 8):
            r, c0 = g // 8, (g % 8) * 16
            iv[slot][r, pl.ds(c0, 16)] = iv[slot][r, pl.ds(c0, 16)] + choff
        for r in range(_ROWS):
            pltpu.async_copy(cube_flat.at[iv[slot].at[r]],
                             hv[slot].at[r], gsem[slot])

    def finish(slot, s, t):
        k, ch = kch(s)
        for r in range(_ROWS):
            pltpu.make_async_copy(cube_flat.at[pl.ds(0, 128)],
                                  hv[slot].at[r], gsem[slot]).wait()

        @pl.when(t > 0)
        def _wait_prev_write():
            pltpu.make_async_copy(sbuf[slot], strips_hbm.at[0, 0],
                                  osem[slot]).wait()

        for g in range(_N_HALO // 16):
            r, c0 = g // 8, (g % 8) * 16
            plsc.store_scatter(sbuf[slot], [drv[r, pl.ds(c0, 16)],
                                            dcv[r, pl.ds(c0, 16)]],
                               hv[slot][r, pl.ds(c0, 16)])
        pltpu.async_copy(sbuf[slot], strips_hbm.at[k, ch], osem[slot])

    for r in range(_R - 1):
        start(r, base_s + r)
        mid(r, base_s + r)

    def body(t, carry):
        s0 = base_s + _R * t
        for r in range(_R):
            s = s0 + r
            sf = s + _R - 1
            slot_f = (r + _R - 1) % _R

            @pl.when(sf < base_s + _PER_W)
            def _fire():
                start(slot_f, sf)
                mid(slot_f, sf)

            finish(r, s, t)
        return carry

    lax.fori_loop(0, n_iter, body, 0)
    pltpu.make_async_copy(sbuf0, strips_hbm.at[0, 0], osem0).wait()
    pltpu.make_async_copy(sbuf1, strips_hbm.at[0, 0], osem1).wait()
    pltpu.make_async_copy(sbuf2, strips_hbm.at[0, 0], osem2).wait()
    pltpu.make_async_copy(sbuf3, strips_hbm.at[0, 0], osem3).wait()


def _tc_body(cube_ref, strip_ref, mask_ref, out_ref):
    mk = mask_ref[0]
    for j in range(_CB):
        cb = cube_ref[0, j]
        st = strip_ref[0, j] * mk
        left = jnp.transpose(st[4:6, 0:_W])
        right = jnp.transpose(st[6:8, 0:_W])
        mid = jnp.concatenate([left, cb, right], axis=1)
        out_ref[0, j] = jnp.concatenate(
            [st[0:2, :], mid, st[2:4, :]], axis=0)


@functools.partial(jax.jit, static_argnums=())
def _sc_tc_pad(bidx, mask_strip, cube, dr, dc):
    mesh = plsc.VectorSubcoreMesh(core_axis_name="c", subcore_axis_name="s",
                                  num_cores=2, num_subcores=16)
    gather_f = pl.kernel(
        _sc_body,
        out_type=jax.ShapeDtypeStruct((_K, _C, 8, _WP), jnp.float32),
        mesh=mesh,
        scratch_types=(
            [pltpu.VMEM((_ROWS, 128), jnp.int32) for _ in range(_R)] +
            [pltpu.VMEM((_ROWS, 128), jnp.float32) for _ in range(_R)] +
            [pltpu.VMEM((8, _WP), jnp.float32) for _ in range(_R)] +
            [pltpu.VMEM((_ROWS, 128), jnp.int32),
             pltpu.VMEM((_ROWS, 128), jnp.int32)] +
            [pltpu.SemaphoreType.DMA for _ in range(3 * _R)]
        ),
        compiler_params=pltpu.CompilerParams(use_tc_tiling_on_sc=True,
                                             needs_layout_passes=False),
    )
    strips = gather_f(bidx, cube.reshape(-1), dr, dc)
    asm = pl.pallas_call(
        _tc_body,
        grid=(_K, _C // _CB),
        in_specs=[
            pl.BlockSpec((1, _CB, _W, _W), lambda k, c: (k, c, 0, 0)),
            pl.BlockSpec((1, _CB, 8, _WP), lambda k, c: (k, c, 0, 0)),
            pl.BlockSpec((1, 8, _WP), lambda k, c: (k, 0, 0)),
        ],
        out_specs=pl.BlockSpec((1, _CB, _WP, _WP), lambda k, c: (k, c, 0, 0)),
        out_shape=jax.ShapeDtypeStruct((_K, _C, _WP, _WP), jnp.float32),
    )
    return asm(cube, strips, mask_strip)


def kernel(cube, to_process, batch_size):
    base, mask_strip = _halo_indices(to_process, batch_size)
    return _sc_tc_pad(base.reshape(_K, _ROWS, 128), mask_strip, cube,
                      jnp.asarray(_SDEST_R), jnp.asarray(_SDEST_C))


# speedup vs baseline: 2.8054x; 1.0173x over previous
import functools

import jax
import jax.numpy as jnp
import numpy as np
from jax import lax
from jax.experimental import pallas as pl
from jax.experimental.pallas import tpu as pltpu
from jax.experimental.pallas import tpu_sc as plsc

_P = 2
_K = 24
_C = 64
_W = 128
_WP = _W + 2 * _P
_N_TB = 2 * _P * _WP
_N_LR = _W * 2 * _P
_N_HALO = _N_TB + _N_LR
_N_PAD = 1152
_ROWS = _N_PAD // 128
_NW = 32
_SLICES = _K * _C
_PER_W = _SLICES // _NW
_CB = 64


def _strip_dest_np():
    rows = np.zeros((_N_PAD,), dtype=np.int32)
    cols = np.arange(_N_PAD, dtype=np.int32) % 128
    s = np.arange(_N_TB)
    rows[:_N_TB] = s // _WP
    cols[:_N_TB] = s % _WP
    t = np.arange(_N_LR)
    rows[_N_TB:_N_HALO] = 4 + t % (2 * _P)
    cols[_N_TB:_N_HALO] = t // (2 * _P)
    return rows.reshape(_ROWS, 128), cols.reshape(_ROWS, 128)


_SDEST_R, _SDEST_C = _strip_dest_np()


def _geometry_np():
    c, w, p = _C, _W, _P
    wp = _WP
    t = ((2.0 * (np.arange(wp, dtype=np.float32) - np.float32(p))
          + np.float32(1.0) - np.float32(w)) / np.float32(w)).astype(np.float32)
    u = np.broadcast_to(t[None, :], (wp, wp)).astype(np.float32)
    v = np.broadcast_to(t[:, None], (wp, wp)).astype(np.float32)
    one = np.ones((wp, wp), dtype=np.float32)
    dirs = np.stack([
        np.stack([one, -v, -u], axis=-1),
        np.stack([-one, -v, u], axis=-1),
        np.stack([u, one, v], axis=-1),
        np.stack([u, -one, -v], axis=-1),
        np.stack([u, -v, one], axis=-1),
        np.stack([-u, -v, -one], axis=-1),
    ], axis=0)
    x, y, z = dirs[..., 0], dirs[..., 1], dirs[..., 2]
    ax, ay, az = np.abs(x), np.abs(y), np.abs(z)
    is_x = (ax >= ay) & (ax >= az)
    is_y = (~is_x) & (ay >= az)
    face = np.where(is_x, np.where(x > 0, 0, 1),
           np.where(is_y, np.where(y > 0, 2, 3),
                    np.where(z > 0, 4, 5)))
    a = np.maximum(np.maximum(ax, ay), az)
    uc = (np.stack([-z, z, x, x, x, -x], axis=0) / a).astype(np.float32)
    vc = (np.stack([-y, -y, z, -z, -y, -y], axis=0) / a).astype(np.float32)
    u2 = np.take_along_axis(uc, face[None], axis=0)[0]
    v2 = np.take_along_axis(vc, face[None], axis=0)[0]
    jj = np.clip(np.floor((u2 + np.float32(1.0)) * np.float32(0.5)
                          * np.float32(w)), 0, w - 1).astype(np.int32)
    ii = np.clip(np.floor((v2 + np.float32(1.0)) * np.float32(0.5)
                          * np.float32(w)), 0, w - 1).astype(np.int32)
    pix = ii * w + jj
    face = face.astype(np.int32)
    pix_tb = np.concatenate([pix[:, :p, :], pix[:, wp - p:, :]], axis=1)
    pix_lr = np.concatenate([pix[:, p:wp - p, :p], pix[:, p:wp - p, wp - p:]], axis=2)
    f_tb = np.concatenate([face[:, :p, :], face[:, wp - p:, :]], axis=1)
    f_lr = np.concatenate([face[:, p:wp - p, :p], face[:, p:wp - p, wp - p:]], axis=2)
    return (pix_tb.reshape(6, _N_TB), pix_lr.reshape(6, _N_LR),
            f_tb.reshape(6, _N_TB), f_lr.reshape(6, _N_LR))


_PT_FLAT, _PL_FLAT, _FT_FLAT, _FL_FLAT = _geometry_np()


def _halo_indices(to_process, batch_size):
    c, w, p = _C, _W, _P
    n_faces = to_process.shape[0]
    bs = n_faces // 6
    bs_delta = jnp.asarray(batch_size, dtype=jnp.int32) - bs
    ar = jnp.arange(n_faces, dtype=jnp.int32)
    inv = jnp.full((n_faces,), -1, dtype=jnp.int32)
    for i in range(n_faces):
        inv = jnp.where(ar == to_process[i], i, inv)
    tpm6 = to_process % 6
    tpd6 = to_process // 6
    idxmat = 6 * (tpd6 + bs_delta)[:, None] + jnp.arange(6, dtype=jnp.int32)[None, :]
    lut = jnp.zeros((n_faces, 6), jnp.int32)
    for d in range(n_faces):
        lut = jnp.where(idxmat == d, inv[d], lut)
    ft_flat = jnp.asarray(_FT_FLAT)
    fl_flat = jnp.asarray(_FL_FLAT)
    pt_flat = jnp.asarray(_PT_FLAT)
    pl_flat = jnp.asarray(_PL_FLAT)
    fsel_tb = jnp.zeros((n_faces, _N_TB), jnp.int32)
    fsel_lr = jnp.zeros((n_faces, _N_LR), jnp.int32)
    ptb = jnp.zeros((n_faces, _N_TB), jnp.int32)
    plr = jnp.zeros((n_faces, _N_LR), jnp.int32)
    for d in range(6):
        sel = (tpm6 == d)[:, None]
        fsel_tb = jnp.where(sel, ft_flat[d][None], fsel_tb)
        fsel_lr = jnp.where(sel, fl_flat[d][None], fsel_lr)
        ptb = jnp.where(sel, pt_flat[d][None], ptb)
        plr = jnp.where(sel, pl_flat[d][None], plr)
    ftb_i = jnp.zeros_like(fsel_tb)
    flr_i = jnp.zeros_like(fsel_lr)
    for j in range(6):
        ftb_i = jnp.where(fsel_tb == j, lut[:, j][:, None], ftb_i)
        flr_i = jnp.where(fsel_lr == j, lut[:, j][:, None], flr_i)
    base_tb = ftb_i * (c * w * w) + ptb
    base_lr = flr_i * (c * w * w) + plr
    m_tb = (ftb_i >= 0)
    m_lr = (flr_i >= 0)
    base = jnp.concatenate([
        jnp.where(m_tb, base_tb, 0),
        jnp.where(m_lr, base_lr, 0),
        jnp.zeros((n_faces, _N_PAD - _N_HALO), jnp.int32),
    ], axis=1)
    mk_tb = m_tb.astype(jnp.float32).reshape(n_faces, 4, _WP)
    mk_lr = jnp.transpose(m_lr.astype(jnp.float32).reshape(n_faces, _W, 4),
                          (0, 2, 1))
    mk_lr = jnp.pad(mk_lr, ((0, 0), (0, 0), (0, 2 * p)))
    mask_strip = jnp.concatenate([mk_tb, mk_lr], axis=1)
    return base, mask_strip


_R = 4


def _sc_body(bidx_hbm, cube_flat, dr_hbm, dc_hbm, strips_hbm,
             iv0, iv1, iv2, iv3, hv0, hv1, hv2, hv3,
             sbuf0, sbuf1, sbuf2, sbuf3, drv, dcv,
             isem0, isem1, isem2, isem3,
             gsem0, gsem1, gsem2, gsem3,
             osem0, osem1, osem2, osem3):
    iv = (iv0, iv1, iv2, iv3)
    hv = (hv0, hv1, hv2, hv3)
    sbuf = (sbuf0, sbuf1, sbuf2, sbuf3)
    isem = (isem0, isem1, isem2, isem3)
    gsem = (gsem0, gsem1, gsem2, gsem3)
    osem = (osem0, osem1, osem2, osem3)
    cid = lax.axis_index("c")
    sid = lax.axis_index("s")
    wid = sid * 2 + cid
    base_s = wid * _PER_W
    pltpu.sync_copy(dr_hbm, drv)
    pltpu.sync_copy(dc_hbm, dcv)
    n_iter = _PER_W // _R

    def kch(s):
        k = s // _C
        return k, s - k * _C

    def start(slot, s):
        k, ch = kch(s)
        pltpu.async_copy(bidx_hbm.at[k], iv[slot], isem[slot])

    def mid(slot, s):
        k, ch = kch(s)
        pltpu.make_async_copy(bidx_hbm.at[0], iv[slot], isem[slot]).wait()
        choff = jnp.zeros((16,), jnp.int32) + ch * (_W * _W)
        for g in range(_ROWS * 8):
            r, c0 = g // 8, (g % 8) * 16
            iv[slot][r, pl.ds(c0, 16)] = iv[slot][r, pl.ds(c0, 16)] + choff
        for r in range(_ROWS):
            pltpu.async_copy(cube_flat.at[iv[slot].at[r]],
                             hv[slot].at[r], gsem[slot])

    def finish(slot, s, t):
        k, ch = kch(s)
        for r in range(_ROWS):
            pltpu.make_async_copy(cube_flat.at[pl.ds(0, 128)],
                                  hv[slot].at[r], gsem[slot]).wait()

        @pl.when(t > 0)
        def _wait_prev_write():
            pltpu.make_async_copy(sbuf[slot], strips_hbm.at[0, 0],
                                  osem[slot]).wait()

        for g in range(_N_HALO // 16):
            r, c0 = g // 8, (g % 8) * 16
            plsc.store_scatter(sbuf[slot], [drv[r, pl.ds(c0, 16)],
                                            dcv[r, pl.ds(c0, 16)]],
                               hv[slot][r, pl.ds(c0, 16)])
        pltpu.async_copy(sbuf[slot], strips_hbm.at[k, ch], osem[slot])

    for r in range(_R - 1):
        start(r, base_s + r)
        mid(r, base_s + r)

    def body(t, carry):
        s0 = base_s + _R * t
        for r in range(_R):
            s = s0 + r
            sf = s + _R - 1
            slot_f = (r + _R - 1) % _R

            @pl.when(sf < base_s + _PER_W)
            def _fire():
                start(slot_f, sf)
                mid(slot_f, sf)

            finish(r, s, t)
        return carry

    lax.fori_loop(0, n_iter, body, 0)
    pltpu.make_async_copy(sbuf0, strips_hbm.at[0, 0], osem0).wait()
    pltpu.make_async_copy(sbuf1, strips_hbm.at[0, 0], osem1).wait()
    pltpu.make_async_copy(sbuf2, strips_hbm.at[0, 0], osem2).wait()
    pltpu.make_async_copy(sbuf3, strips_hbm.at[0, 0], osem3).wait()


def _tc_body(cube_ref, strip_ref, mask_ref, out_ref):
    mk = mask_ref[0]
    for j in range(_CB):
        cb = cube_ref[0, j]
        st = strip_ref[0, j] * mk
        left = jnp.transpose(st[4:6, 0:_W])
        right = jnp.transpose(st[6:8, 0:_W])
        mid = jnp.concatenate([left, cb, right], axis=1)
        out_ref[0, j] = jnp.concatenate(
            [st[0:2, :], mid, st[2:4, :]], axis=0)


@functools.partial(jax.jit, static_argnums=())
def _sc_tc_pad(bidx, mask_strip, cube, dr, dc):
    mesh = plsc.VectorSubcoreMesh(core_axis_name="c", subcore_axis_name="s",
                                  num_cores=2, num_subcores=16)
    gather_f = pl.kernel(
        _sc_body,
        out_type=jax.ShapeDtypeStruct((_K, _C, 8, _WP), jnp.float32),
        mesh=mesh,
        scratch_types=(
            [pltpu.VMEM((_ROWS, 128), jnp.int32) for _ in range(_R)] +
            [pltpu.VMEM((_ROWS, 128), jnp.float32) for _ in range(_R)] +
            [pltpu.VMEM((8, _WP), jnp.float32) for _ in range(_R)] +
            [pltpu.VMEM((_ROWS, 128), jnp.int32),
             pltpu.VMEM((_ROWS, 128), jnp.int32)] +
            [pltpu.SemaphoreType.DMA for _ in range(3 * _R)]
        ),
        compiler_params=pltpu.CompilerParams(use_tc_tiling_on_sc=True,
                                             needs_layout_passes=False),
    )
    strips = gather_f(bidx, cube.reshape(-1), dr, dc)
    asm = pl.pallas_call(
        _tc_body,
        grid=(_K, _C // _CB),
        in_specs=[
            pl.BlockSpec((1, _CB, _W, _W), lambda k, c: (k, c, 0, 0)),
            pl.BlockSpec((1, _CB, 8, _WP), lambda k, c: (k, c, 0, 0)),
            pl.BlockSpec((1, 8, _WP), lambda k, c: (k, 0, 0)),
        ],
        out_specs=pl.BlockSpec((1, _CB, _WP, _WP), lambda k, c: (k, c, 0, 0)),
        out_shape=jax.ShapeDtypeStruct((_K, _C, _WP, _WP), jnp.float32),
    )
    return asm(cube, strips, mask_strip)


def kernel(cube, to_process, batch_size):
    base, mask_strip = _halo_indices(to_process, batch_size)
    return _sc_tc_pad(base.reshape(_K, _ROWS, 128), mask_strip, cube,
                      jnp.asarray(_SDEST_R), jnp.asarray(_SDEST_C))
